# Initial kernel scaffold; baseline (speedup 1.0000x reference)
import jax
import jax.numpy as jnp
from jax.experimental import pallas as pl


def kernel(node_feats, edge_attr, edge_index, Wsg, bsg, Wdg, bdg, Weg, beg, g1, b1, Wsu, bsu, Wdu, bdu, g2, b2):
    raise NotImplementedError("write your pallas kernel here")


# speedup vs baseline: 1.9918x; 1.9918x over previous
import functools

import jax
import jax.numpy as jnp
from jax import lax
from jax.experimental import pallas as pl
from jax.experimental.pallas import tpu as pltpu
from jax.experimental.pallas import tpu_sc as plsc

N = 10000
E = 320000
D = 128

NC = 2
NS = 16
NW = NC * NS
CH = 80
EPW = E // NW
ACC_ROWS = 5120
HALF = N // 2

BN_BLK = 400
BE_BLK = 1600

_f32 = jnp.float32


def _dotT(x, w):
    return lax.dot_general(x, w, (((1,), (1,)), ((), ())),
                           preferred_element_type=_f32)


def _tables_body(x_ref, wsg_ref, wdu_ref, wdg_ref, wsu_ref, p_ref,
                 g_ref, adg_ref, asu_ref):
    x = x_ref[...]
    g_ref[:, :D] = _dotT(x, wsg_ref[...])
    g_ref[:, D:] = _dotT(x, wdu_ref[...]) + p_ref[0][None, :]
    adg_ref[...] = _dotT(x, wdg_ref[...])
    asu_ref[...] = _dotT(x, wsu_ref[...])


def _edge_pre_body(ea_ref, ga_ref, gb_ref, weg_ref, pre_ref, st_ref):
    i = pl.program_id(0)
    pre = _dotT(ea_ref[...], weg_ref[...]) + ga_ref[...] + gb_ref[...]
    pre_ref[...] = pre
    s1 = jnp.sum(pre, axis=0)
    s2 = jnp.sum(pre * pre, axis=0)
    blk = jnp.concatenate([s1[None], s2[None], jnp.zeros((6, D), _f32)], 0)

    @pl.when(i == 0)
    def _():
        st_ref[...] = blk

    @pl.when(i > 0)
    def _():
        st_ref[...] += blk


def _edge_fin_body(pre_ref, ea_ref, adu_ref, st_ref, p_ref,
                   nea_ref, sig_ref, msg_ref):
    st = st_ref[...]
    mean = st[0] / E
    var = st[1] / E - mean * mean
    inv = lax.rsqrt(var + 1e-5)
    xh = (pre_ref[...] - mean[None, :]) * inv[None, :] * p_ref[0][None, :] \
        + p_ref[1][None, :]
    nea = ea_ref[...] + xh * jax.nn.sigmoid(xh)
    sig = jax.nn.sigmoid(nea)
    nea_ref[...] = nea
    sig_ref[...] = sig
    msg_ref[...] = sig * adu_ref[...]


def _node_pre_body(asu_ref, nag_ref, eag_ref, pre_ref, st_ref):
    i = pl.program_id(0)
    pre = asu_ref[...] + nag_ref[...] / (eag_ref[...] + 1e-6)
    pre_ref[...] = pre
    s1 = jnp.sum(pre, axis=0)
    s2 = jnp.sum(pre * pre, axis=0)
    blk = jnp.concatenate([s1[None], s2[None], jnp.zeros((6, D), _f32)], 0)

    @pl.when(i == 0)
    def _():
        st_ref[...] = blk

    @pl.when(i > 0)
    def _():
        st_ref[...] += blk


def _node_fin_body(x_ref, pre_ref, st_ref, p_ref, out_ref):
    st = st_ref[...]
    mean = st[0] / N
    var = st[1] / N - mean * mean
    inv = lax.rsqrt(var + 1e-5)
    xh = (pre_ref[...] - mean[None, :]) * inv[None, :] * p_ref[0][None, :] \
        + p_ref[1][None, :]
    out_ref[...] = x_ref[...] + xh * jax.nn.sigmoid(xh)


def _sc_mesh():
    return plsc.VectorSubcoreMesh(core_axis_name="c", subcore_axis_name="s",
                                  num_cores=NC, num_subcores=NS)


@functools.cache
def _build_sc_gather():
    return functools.partial(
        pl.kernel,
        out_type=(jax.ShapeDtypeStruct((E, 2 * D), _f32),
                  jax.ShapeDtypeStruct((E, D), _f32)),
        mesh=_sc_mesh(),
        scratch_types=[
            pltpu.VMEM((CH,), jnp.int32),
            pltpu.VMEM((CH,), jnp.int32),
            pltpu.VMEM((CH, 2 * D), _f32),
            pltpu.VMEM((CH, D), _f32),
            pltpu.SemaphoreType.DMA,
            pltpu.SemaphoreType.DMA,
        ],
    )(_sc_gather_body)


def _sc_gather(g_tab, adg, row, col):
    return _build_sc_gather()(g_tab, adg, row, col)


def _sc_gather_body(g_hbm, adg_hbm, row_hbm, col_hbm, ga_hbm, gb_hbm,
                    idr, idc, gbuf, bbuf, sem1, sem2):
    wid = lax.axis_index("s") * NC + lax.axis_index("c")

    def body(k, carry):
        base = wid * EPW + k * CH
        pltpu.sync_copy(row_hbm.at[pl.ds(base, CH)], idr)
        pltpu.sync_copy(col_hbm.at[pl.ds(base, CH)], idc)
        cp1 = pltpu.async_copy(g_hbm.at[idr], gbuf, sem1)
        cp2 = pltpu.async_copy(adg_hbm.at[idc], bbuf, sem2)
        cp1.wait()
        cp2.wait()
        pltpu.sync_copy(gbuf, ga_hbm.at[pl.ds(base, CH)])
        pltpu.sync_copy(bbuf, gb_hbm.at[pl.ds(base, CH)])
        return carry

    lax.fori_loop(0, EPW // CH, body, 0)


@functools.cache
def _build_sc_scatter():
    return functools.partial(
        pl.kernel,
        out_type=(jax.ShapeDtypeStruct((N, D), _f32),
                  jax.ShapeDtypeStruct((N, D), _f32)),
        mesh=_sc_mesh(),
        scratch_types=[
            pltpu.VMEM_SHARED((ACC_ROWS, D), _f32),
            pltpu.VMEM_SHARED((ACC_ROWS, D), _f32),
            pltpu.VMEM((16, D), _f32),
            pltpu.VMEM((CH,), jnp.int32),
            pltpu.VMEM((8, CH), jnp.int32),
            pltpu.VMEM((CH, D), _f32),
            pltpu.VMEM((CH, D), _f32),
        ],
    )(_sc_scatter_body)


def _sc_scatter(sig, msg, col):
    return _build_sc_scatter()(sig, msg, col)


def _sc_scatter_body(sig_hbm, msg_hbm, col_hbm, nag_hbm, eag_hbm,
                     acc_n, acc_e, zb, idc, lidx, sbuf, mbuf):
    c = lax.axis_index("c")
    s = lax.axis_index("s")
    rows_per_tile = ACC_ROWS // NS

    zv = jnp.zeros((16,), _f32)
    for i in range(16):
        for g in range(D // 16):
            zb[i, pl.ds(g * 16, 16)] = zv

    def zbody(k, carry):
        off = s * rows_per_tile + k * 16
        pltpu.sync_copy(zb, acc_n.at[pl.ds(off, 16)])
        pltpu.sync_copy(zb, acc_e.at[pl.ds(off, 16)])
        return carry

    lax.fori_loop(0, rows_per_tile // 16, zbody, 0)
    plsc.subcore_barrier()

    nbase = c * HALF
    epw = E // NS

    def body(k, carry):
        eb = s * epw + k * CH
        pltpu.sync_copy(col_hbm.at[pl.ds(eb, CH)], idc)
        pltpu.sync_copy(sig_hbm.at[pl.ds(eb, CH)], sbuf)
        pltpu.sync_copy(msg_hbm.at[pl.ds(eb, CH)], mbuf)
        for g in range(CH // 16):
            v = idc[pl.ds(g * 16, 16)]
            loc = v - nbase
            ok = (loc >= 0) & (loc < HALF)
            lidx[0, pl.ds(g * 16, 16)] = jnp.where(ok, loc, HALF)
        pltpu.sync_copy(sbuf, acc_e.at[lidx.at[0]], add=True)
        pltpu.sync_copy(mbuf, acc_n.at[lidx.at[0]], add=True)
        return carry

    lax.fori_loop(0, epw // CH, body, 0)
    plsc.subcore_barrier()

    tail = HALF - 15 * rows_per_tile

    @pl.when(s < NS - 1)
    def _():
        off = s * rows_per_tile
        pltpu.sync_copy(acc_n.at[pl.ds(off, rows_per_tile)],
                        nag_hbm.at[pl.ds(nbase + off, rows_per_tile)])
        pltpu.sync_copy(acc_e.at[pl.ds(off, rows_per_tile)],
                        eag_hbm.at[pl.ds(nbase + off, rows_per_tile)])

    @pl.when(s == NS - 1)
    def _():
        off = (NS - 1) * rows_per_tile
        pltpu.sync_copy(acc_n.at[pl.ds(off, tail)],
                        nag_hbm.at[pl.ds(nbase + off, tail)])
        pltpu.sync_copy(acc_e.at[pl.ds(off, tail)],
                        eag_hbm.at[pl.ds(nbase + off, tail)])


def kernel(node_feats, edge_attr, edge_index, Wsg, bsg, Wdg, bdg, Weg, beg,
           g1, b1, Wsu, bsu, Wdu, bdu, g2, b2):
    del bsg, bdg, beg, bsu
    row = edge_index[0]
    col = edge_index[1]
    p_tab = jnp.concatenate([bdu[None], jnp.zeros((7, D), _f32)], 0)
    p_e = jnp.concatenate([g1[None], b1[None], jnp.zeros((6, D), _f32)], 0)
    p_n = jnp.concatenate([g2[None], b2[None], jnp.zeros((6, D), _f32)], 0)

    nb = N // BN_BLK
    eb = E // BE_BLK

    g_tab, adg, asu = pl.pallas_call(
        _tables_body,
        grid=(nb,),
        in_specs=[
            pl.BlockSpec((BN_BLK, D), lambda i: (i, 0)),
            pl.BlockSpec((D, D), lambda i: (0, 0)),
            pl.BlockSpec((D, D), lambda i: (0, 0)),
            pl.BlockSpec((D, D), lambda i: (0, 0)),
            pl.BlockSpec((D, D), lambda i: (0, 0)),
            pl.BlockSpec((8, D), lambda i: (0, 0)),
        ],
        out_specs=[
            pl.BlockSpec((BN_BLK, 2 * D), lambda i: (i, 0)),
            pl.BlockSpec((BN_BLK, D), lambda i: (i, 0)),
            pl.BlockSpec((BN_BLK, D), lambda i: (i, 0)),
        ],
        out_shape=[
            jax.ShapeDtypeStruct((N, 2 * D), _f32),
            jax.ShapeDtypeStruct((N, D), _f32),
            jax.ShapeDtypeStruct((N, D), _f32),
        ],
    )(node_feats, Wsg, Wdu, Wdg, Wsu, p_tab)

    ga, gb = _sc_gather(g_tab, adg, row, col)

    pre_e, st1 = pl.pallas_call(
        _edge_pre_body,
        grid=(eb,),
        in_specs=[
            pl.BlockSpec((BE_BLK, D), lambda i: (i, 0)),
            pl.BlockSpec((BE_BLK, D), lambda i: (i, 0)),
            pl.BlockSpec((BE_BLK, D), lambda i: (i, 0)),
            pl.BlockSpec((D, D), lambda i: (0, 0)),
        ],
        out_specs=[
            pl.BlockSpec((BE_BLK, D), lambda i: (i, 0)),
            pl.BlockSpec((8, D), lambda i: (0, 0)),
        ],
        out_shape=[
            jax.ShapeDtypeStruct((E, D), _f32),
            jax.ShapeDtypeStruct((8, D), _f32),
        ],
    )(edge_attr, ga, gb, Weg)

    nea, sig, msg = pl.pallas_call(
        _edge_fin_body,
        grid=(eb,),
        in_specs=[
            pl.BlockSpec((BE_BLK, D), lambda i: (i, 0)),
            pl.BlockSpec((BE_BLK, D), lambda i: (i, 0)),
            pl.BlockSpec((BE_BLK, D), lambda i: (i, 1)),
            pl.BlockSpec((8, D), lambda i: (0, 0)),
            pl.BlockSpec((8, D), lambda i: (0, 0)),
        ],
        out_specs=[
            pl.BlockSpec((BE_BLK, D), lambda i: (i, 0)),
            pl.BlockSpec((BE_BLK, D), lambda i: (i, 0)),
            pl.BlockSpec((BE_BLK, D), lambda i: (i, 0)),
        ],
        out_shape=[
            jax.ShapeDtypeStruct((E, D), _f32),
            jax.ShapeDtypeStruct((E, D), _f32),
            jax.ShapeDtypeStruct((E, D), _f32),
        ],
    )(pre_e, edge_attr, ga, st1, p_e)

    nagg, eagg = _sc_scatter(sig, msg, col)

    pre_n, st2 = pl.pallas_call(
        _node_pre_body,
        grid=(nb,),
        in_specs=[
            pl.BlockSpec((BN_BLK, D), lambda i: (i, 0)),
            pl.BlockSpec((BN_BLK, D), lambda i: (i, 0)),
            pl.BlockSpec((BN_BLK, D), lambda i: (i, 0)),
        ],
        out_specs=[
            pl.BlockSpec((BN_BLK, D), lambda i: (i, 0)),
            pl.BlockSpec((8, D), lambda i: (0, 0)),
        ],
        out_shape=[
            jax.ShapeDtypeStruct((N, D), _f32),
            jax.ShapeDtypeStruct((8, D), _f32),
        ],
    )(asu, nagg, eagg)

    new_node_feats = pl.pallas_call(
        _node_fin_body,
        grid=(nb,),
        in_specs=[
            pl.BlockSpec((BN_BLK, D), lambda i: (i, 0)),
            pl.BlockSpec((BN_BLK, D), lambda i: (i, 0)),
            pl.BlockSpec((8, D), lambda i: (0, 0)),
            pl.BlockSpec((8, D), lambda i: (0, 0)),
        ],
        out_specs=pl.BlockSpec((BN_BLK, D), lambda i: (i, 0)),
        out_shape=jax.ShapeDtypeStruct((N, D), _f32),
    )(node_feats, pre_n, st2, p_n)

    return (new_node_feats, nea)


# speedup vs baseline: 2.4235x; 1.2168x over previous
import functools

import jax
import jax.numpy as jnp
from jax import lax
from jax.experimental import pallas as pl
from jax.experimental.pallas import tpu as pltpu
from jax.experimental.pallas import tpu_sc as plsc

N = 10000
E = 320000
D = 128

NC = 2
NS = 16
NW = NC * NS
CH = 80
EPW = E // NW
ACC_ROWS = 10240
ROWS_PER_TILE = ACC_ROWS // NS

BN_BLK = 400
BE_BLK = 1600

_f32 = jnp.float32


def _dotT(x, w):
    return lax.dot_general(x, w, (((1,), (1,)), ((), ())),
                           preferred_element_type=_f32)


def _tables_body(x_ref, wsg_ref, wdu_ref, wdg_ref, wsu_ref, p_ref,
                 asg_ref, adu_ref, adg_ref, asu_ref):
    x = x_ref[...]
    asg_ref[...] = _dotT(x, wsg_ref[...])
    adu_ref[...] = _dotT(x, wdu_ref[...]) + p_ref[0][None, :]
    adg_ref[...] = _dotT(x, wdg_ref[...])
    asu_ref[...] = _dotT(x, wsu_ref[...])


def _edge_pre_body(ea_ref, asum_ref, weg_ref, pre_ref, st_ref):
    i = pl.program_id(0)
    pre = _dotT(ea_ref[...], weg_ref[...]) + asum_ref[...]
    pre_ref[...] = pre
    s1 = jnp.sum(pre, axis=0)
    s2 = jnp.sum(pre * pre, axis=0)
    blk = jnp.concatenate([s1[None], s2[None], jnp.zeros((6, D), _f32)], 0)

    @pl.when(i == 0)
    def _():
        st_ref[...] = blk

    @pl.when(i > 0)
    def _():
        st_ref[...] += blk


def _edge_fin_body(pre_ref, ea_ref, au_ref, st_ref, p_ref,
                   nea_ref, sig_ref, msg_ref):
    st = st_ref[...]
    mean = st[0] / E
    var = st[1] / E - mean * mean
    inv = lax.rsqrt(var + 1e-5)
    xh = (pre_ref[...] - mean[None, :]) * inv[None, :] * p_ref[0][None, :] \
        + p_ref[1][None, :]
    nea = ea_ref[...] + xh * jax.nn.sigmoid(xh)
    sig = jax.nn.sigmoid(nea)
    nea_ref[...] = nea
    sig_ref[...] = sig
    msg_ref[...] = sig * au_ref[...]


def _node_pre_body(asu_ref, nag_ref, eag_ref, pre_ref, st_ref):
    i = pl.program_id(0)
    pre = asu_ref[...] + nag_ref[...] / (eag_ref[...] + 1e-6)
    pre_ref[...] = pre
    s1 = jnp.sum(pre, axis=0)
    s2 = jnp.sum(pre * pre, axis=0)
    blk = jnp.concatenate([s1[None], s2[None], jnp.zeros((6, D), _f32)], 0)

    @pl.when(i == 0)
    def _():
        st_ref[...] = blk

    @pl.when(i > 0)
    def _():
        st_ref[...] += blk


def _node_fin_body(x_ref, pre_ref, st_ref, p_ref, out_ref):
    st = st_ref[...]
    mean = st[0] / N
    var = st[1] / N - mean * mean
    inv = lax.rsqrt(var + 1e-5)
    xh = (pre_ref[...] - mean[None, :]) * inv[None, :] * p_ref[0][None, :] \
        + p_ref[1][None, :]
    out_ref[...] = x_ref[...] + xh * jax.nn.sigmoid(xh)


def _sc_mesh():
    return plsc.VectorSubcoreMesh(core_axis_name="c", subcore_axis_name="s",
                                  num_cores=NC, num_subcores=NS)


@functools.cache
def _build_sc_gather():
    return functools.partial(
        pl.kernel,
        out_type=(jax.ShapeDtypeStruct((E, D), _f32),
                  jax.ShapeDtypeStruct((E, D), _f32)),
        mesh=_sc_mesh(),
        scratch_types=[
            pltpu.VMEM((CH,), jnp.int32),
            pltpu.VMEM((CH,), jnp.int32),
            pltpu.VMEM((CH, D), _f32),
            pltpu.VMEM((CH, D), _f32),
            pltpu.VMEM((CH, D), _f32),
            pltpu.SemaphoreType.DMA,
            pltpu.SemaphoreType.DMA,
            pltpu.SemaphoreType.DMA,
        ],
    )(_sc_gather_body)


def _sc_gather(asg, adu, adg, row, col):
    return _build_sc_gather()(asg, adu, adg, row, col)


def _sc_gather_body(asg_hbm, adu_hbm, adg_hbm, row_hbm, col_hbm,
                    asum_hbm, au_hbm, idr, idc, abuf, ubuf, bbuf,
                    sem1, sem2, sem3):
    wid = lax.axis_index("s") * NC + lax.axis_index("c")

    def body(k, carry):
        base = wid * EPW + k * CH
        pltpu.sync_copy(row_hbm.at[pl.ds(base, CH)], idr)
        pltpu.sync_copy(col_hbm.at[pl.ds(base, CH)], idc)
        cp1 = pltpu.async_copy(asg_hbm.at[idr], abuf, sem1)
        cp2 = pltpu.async_copy(adu_hbm.at[idr], ubuf, sem2)
        cp3 = pltpu.async_copy(adg_hbm.at[idc], bbuf, sem3)
        cp1.wait()
        cp3.wait()

        def rowbody(i, carry2):
            for g in range(D // 16):
                sl = pl.ds(g * 16, 16)
                abuf[i, sl] = abuf[i, sl] + bbuf[i, sl]
            return carry2

        lax.fori_loop(0, CH, rowbody, 0)
        cp2.wait()
        pltpu.sync_copy(abuf, asum_hbm.at[pl.ds(base, CH)])
        pltpu.sync_copy(ubuf, au_hbm.at[pl.ds(base, CH)])
        return carry

    lax.fori_loop(0, EPW // CH, body, 0)


@functools.cache
def _build_sc_scatter():
    return functools.partial(
        pl.kernel,
        out_type=(jax.ShapeDtypeStruct((N, D), _f32),
                  jax.ShapeDtypeStruct((N, D), _f32)),
        mesh=_sc_mesh(),
        scratch_types=[
            pltpu.VMEM_SHARED((ACC_ROWS, D), _f32),
            pltpu.VMEM((16, D), _f32),
            pltpu.VMEM((8, CH), jnp.int32),
            pltpu.VMEM((CH, D), _f32),
        ],
    )(_sc_scatter_body)


def _sc_scatter(sig, msg, col):
    return _build_sc_scatter()(sig, msg, col)


def _sc_scatter_body(sig_hbm, msg_hbm, col_hbm, nag_hbm, eag_hbm,
                     acc, zb, lidx, dbuf):
    c = lax.axis_index("c")
    s = lax.axis_index("s")

    zv = jnp.zeros((16,), _f32)
    for i in range(16):
        for g in range(D // 16):
            zb[i, pl.ds(g * 16, 16)] = zv

    def zbody(k, carry):
        pltpu.sync_copy(zb, acc.at[pl.ds(s * ROWS_PER_TILE + k * 16, 16)])
        return carry

    lax.fori_loop(0, ROWS_PER_TILE // 16, zbody, 0)
    plsc.subcore_barrier()

    epw = E // NS

    def make_loop(data_hbm):
        def body(k, carry):
            eb = s * epw + k * CH
            pltpu.sync_copy(col_hbm.at[pl.ds(eb, CH)], lidx.at[0])
            pltpu.sync_copy(data_hbm.at[pl.ds(eb, CH)], dbuf)
            pltpu.sync_copy(dbuf, acc.at[lidx.at[0]], add=True)
            return carry
        return body

    @pl.when(c == 0)
    def _():
        lax.fori_loop(0, epw // CH, make_loop(sig_hbm), 0)

    @pl.when(c == 1)
    def _():
        lax.fori_loop(0, epw // CH, make_loop(msg_hbm), 0)

    plsc.subcore_barrier()

    tail = N - (NS - 1) * ROWS_PER_TILE

    def dump(out_hbm):
        off = s * ROWS_PER_TILE

        @pl.when(s < NS - 1)
        def _():
            pltpu.sync_copy(acc.at[pl.ds(off, ROWS_PER_TILE)],
                            out_hbm.at[pl.ds(off, ROWS_PER_TILE)])

        @pl.when(s == NS - 1)
        def _():
            pltpu.sync_copy(acc.at[pl.ds(off, tail)],
                            out_hbm.at[pl.ds(off, tail)])

    @pl.when(c == 0)
    def _():
        dump(eag_hbm)

    @pl.when(c == 1)
    def _():
        dump(nag_hbm)


def kernel(node_feats, edge_attr, edge_index, Wsg, bsg, Wdg, bdg, Weg, beg,
           g1, b1, Wsu, bsu, Wdu, bdu, g2, b2):
    del bsg, bdg, beg, bsu
    row = edge_index[0]
    col = edge_index[1]
    p_tab = jnp.concatenate([bdu[None], jnp.zeros((7, D), _f32)], 0)
    p_e = jnp.concatenate([g1[None], b1[None], jnp.zeros((6, D), _f32)], 0)
    p_n = jnp.concatenate([g2[None], b2[None], jnp.zeros((6, D), _f32)], 0)

    nb = N // BN_BLK
    eb = E // BE_BLK

    asg, adu, adg, asu = pl.pallas_call(
        _tables_body,
        grid=(nb,),
        in_specs=[
            pl.BlockSpec((BN_BLK, D), lambda i: (i, 0)),
            pl.BlockSpec((D, D), lambda i: (0, 0)),
            pl.BlockSpec((D, D), lambda i: (0, 0)),
            pl.BlockSpec((D, D), lambda i: (0, 0)),
            pl.BlockSpec((D, D), lambda i: (0, 0)),
            pl.BlockSpec((8, D), lambda i: (0, 0)),
        ],
        out_specs=[
            pl.BlockSpec((BN_BLK, D), lambda i: (i, 0)),
            pl.BlockSpec((BN_BLK, D), lambda i: (i, 0)),
            pl.BlockSpec((BN_BLK, D), lambda i: (i, 0)),
            pl.BlockSpec((BN_BLK, D), lambda i: (i, 0)),
        ],
        out_shape=[
            jax.ShapeDtypeStruct((N, D), _f32),
            jax.ShapeDtypeStruct((N, D), _f32),
            jax.ShapeDtypeStruct((N, D), _f32),
            jax.ShapeDtypeStruct((N, D), _f32),
        ],
    )(node_feats, Wsg, Wdu, Wdg, Wsu, p_tab)

    asum, au = _sc_gather(asg, adu, adg, row, col)

    pre_e, st1 = pl.pallas_call(
        _edge_pre_body,
        grid=(eb,),
        in_specs=[
            pl.BlockSpec((BE_BLK, D), lambda i: (i, 0)),
            pl.BlockSpec((BE_BLK, D), lambda i: (i, 0)),
            pl.BlockSpec((D, D), lambda i: (0, 0)),
        ],
        out_specs=[
            pl.BlockSpec((BE_BLK, D), lambda i: (i, 0)),
            pl.BlockSpec((8, D), lambda i: (0, 0)),
        ],
        out_shape=[
            jax.ShapeDtypeStruct((E, D), _f32),
            jax.ShapeDtypeStruct((8, D), _f32),
        ],
    )(edge_attr, asum, Weg)

    nea, sig, msg = pl.pallas_call(
        _edge_fin_body,
        grid=(eb,),
        in_specs=[
            pl.BlockSpec((BE_BLK, D), lambda i: (i, 0)),
            pl.BlockSpec((BE_BLK, D), lambda i: (i, 0)),
            pl.BlockSpec((BE_BLK, D), lambda i: (i, 0)),
            pl.BlockSpec((8, D), lambda i: (0, 0)),
            pl.BlockSpec((8, D), lambda i: (0, 0)),
        ],
        out_specs=[
            pl.BlockSpec((BE_BLK, D), lambda i: (i, 0)),
            pl.BlockSpec((BE_BLK, D), lambda i: (i, 0)),
            pl.BlockSpec((BE_BLK, D), lambda i: (i, 0)),
        ],
        out_shape=[
            jax.ShapeDtypeStruct((E, D), _f32),
            jax.ShapeDtypeStruct((E, D), _f32),
            jax.ShapeDtypeStruct((E, D), _f32),
        ],
    )(pre_e, edge_attr, au, st1, p_e)

    nagg, eagg = _sc_scatter(sig, msg, col)

    pre_n, st2 = pl.pallas_call(
        _node_pre_body,
        grid=(nb,),
        in_specs=[
            pl.BlockSpec((BN_BLK, D), lambda i: (i, 0)),
            pl.BlockSpec((BN_BLK, D), lambda i: (i, 0)),
            pl.BlockSpec((BN_BLK, D), lambda i: (i, 0)),
        ],
        out_specs=[
            pl.BlockSpec((BN_BLK, D), lambda i: (i, 0)),
            pl.BlockSpec((8, D), lambda i: (0, 0)),
        ],
        out_shape=[
            jax.ShapeDtypeStruct((N, D), _f32),
            jax.ShapeDtypeStruct((8, D), _f32),
        ],
    )(asu, nagg, eagg)

    new_node_feats = pl.pallas_call(
        _node_fin_body,
        grid=(nb,),
        in_specs=[
            pl.BlockSpec((BN_BLK, D), lambda i: (i, 0)),
            pl.BlockSpec((BN_BLK, D), lambda i: (i, 0)),
            pl.BlockSpec((8, D), lambda i: (0, 0)),
            pl.BlockSpec((8, D), lambda i: (0, 0)),
        ],
        out_specs=pl.BlockSpec((BN_BLK, D), lambda i: (i, 0)),
        out_shape=jax.ShapeDtypeStruct((N, D), _f32),
    )(node_feats, pre_n, st2, p_n)

    return (new_node_feats, nea)


# speedup vs baseline: 2.6447x; 1.0912x over previous
import functools

import jax
import jax.numpy as jnp
from jax import lax
from jax.experimental import pallas as pl
from jax.experimental.pallas import tpu as pltpu
from jax.experimental.pallas import tpu_sc as plsc

N = 10000
E = 320000
D = 128

NC = 2
NS = 16
NW = NC * NS
CH = 80
EPW = E // NW
ACC_ROWS = 10240
ROWS_PER_TILE = ACC_ROWS // NS

BN_BLK = 400
BE_BLK = 1600

_f32 = jnp.float32
_bf16 = jnp.bfloat16


def _dotT(x, w):
    return lax.dot_general(x, w, (((1,), (1,)), ((), ())),
                           preferred_element_type=_f32)


def _tables_body(x_ref, wsg_ref, wdu_ref, wdg_ref, wsu_ref, p_ref,
                 gau_ref, adg_ref, asu_ref):
    x = x_ref[...]
    gau_ref[:, :D] = _dotT(x, wsg_ref[...])
    gau_ref[:, D:] = _dotT(x, wdu_ref[...]) + p_ref[0][None, :]
    adg_ref[...] = _dotT(x, wdg_ref[...])
    asu_ref[...] = _dotT(x, wsu_ref[...])


def _edge_pre_body(ea_ref, ga_ref, gb_ref, weg_ref, pre_ref, st_ref):
    i = pl.program_id(0)
    pre = _dotT(ea_ref[...], weg_ref[...]) + ga_ref[...] + gb_ref[...]
    pre_ref[...] = pre
    s1 = jnp.sum(pre, axis=0)
    s2 = jnp.sum(pre * pre, axis=0)
    blk = jnp.concatenate([s1[None], s2[None], jnp.zeros((6, D), _f32)], 0)

    @pl.when(i == 0)
    def _():
        st_ref[...] = blk

    @pl.when(i > 0)
    def _():
        st_ref[...] += blk


def _edge_fin_body(pre_ref, ea_ref, au_ref, st_ref, p_ref,
                   nea_ref, sig_ref, msg_ref):
    st = st_ref[...]
    mean = st[0] / E
    var = st[1] / E - mean * mean
    inv = lax.rsqrt(var + 1e-5)
    xh = (pre_ref[...] - mean[None, :]) * inv[None, :] * p_ref[0][None, :] \
        + p_ref[1][None, :]
    nea = ea_ref[...] + xh * jax.nn.sigmoid(xh)
    sig = jax.nn.sigmoid(nea)
    nea_ref[...] = nea
    sig_ref[...] = sig
    msg_ref[...] = sig * au_ref[...]


def _node_pre_body(asu_ref, nag_ref, eag_ref, pre_ref, st_ref):
    i = pl.program_id(0)
    pre = asu_ref[...] + nag_ref[...] / (eag_ref[...] + 1e-6)
    pre_ref[...] = pre
    s1 = jnp.sum(pre, axis=0)
    s2 = jnp.sum(pre * pre, axis=0)
    blk = jnp.concatenate([s1[None], s2[None], jnp.zeros((6, D), _f32)], 0)

    @pl.when(i == 0)
    def _():
        st_ref[...] = blk

    @pl.when(i > 0)
    def _():
        st_ref[...] += blk


def _node_fin_body(x_ref, pre_ref, st_ref, p_ref, out_ref):
    st = st_ref[...]
    mean = st[0] / N
    var = st[1] / N - mean * mean
    inv = lax.rsqrt(var + 1e-5)
    xh = (pre_ref[...] - mean[None, :]) * inv[None, :] * p_ref[0][None, :] \
        + p_ref[1][None, :]
    out_ref[...] = x_ref[...] + xh * jax.nn.sigmoid(xh)


def _sc_mesh():
    return plsc.VectorSubcoreMesh(core_axis_name="c", subcore_axis_name="s",
                                  num_cores=NC, num_subcores=NS)


@functools.cache
def _build_sc_gather():
    return functools.partial(
        pl.kernel,
        out_type=(jax.ShapeDtypeStruct((E, 2 * D), _f32),
                  jax.ShapeDtypeStruct((E, D), _f32)),
        mesh=_sc_mesh(),
        scratch_types=[
            pltpu.VMEM((2, CH), jnp.int32),
            pltpu.VMEM((2, CH), jnp.int32),
            pltpu.VMEM((2, CH, 2 * D), _f32),
            pltpu.VMEM((2, CH, D), _f32),
            pltpu.SemaphoreType.DMA,
            pltpu.SemaphoreType.DMA,
            pltpu.SemaphoreType.DMA,
            pltpu.SemaphoreType.DMA,
        ],
    )(_sc_gather_body)


def _sc_gather(gtab, adg, row, col):
    return _build_sc_gather()(gtab, adg, row, col)


def _sc_gather_body(gtab_hbm, adg_hbm, row_hbm, col_hbm,
                    gau_hbm, gb_hbm, idr, idc, abuf, bbuf, *sems):
    wid = lax.axis_index("s") * NC + lax.axis_index("c")
    nch = EPW // CH

    def start(k, slot):
        base = wid * EPW + k * CH
        pltpu.sync_copy(row_hbm.at[pl.ds(base, CH)], idr.at[slot])
        pltpu.sync_copy(col_hbm.at[pl.ds(base, CH)], idc.at[slot])
        cp1 = pltpu.async_copy(gtab_hbm.at[idr.at[slot]], abuf.at[slot],
                               sems[2 * slot + 0])
        cp2 = pltpu.async_copy(adg_hbm.at[idc.at[slot]], bbuf.at[slot],
                               sems[2 * slot + 1])
        return cp1, cp2

    def finish(k, slot, cps):
        base = wid * EPW + k * CH
        cps[0].wait()
        pltpu.sync_copy(abuf.at[slot], gau_hbm.at[pl.ds(base, CH)])
        cps[1].wait()
        pltpu.sync_copy(bbuf.at[slot], gb_hbm.at[pl.ds(base, CH)])

    def body(j, carry):
        k = j * 2
        cps0 = start(k, 0)
        cps1 = start(k + 1, 1)
        finish(k, 0, cps0)
        finish(k + 1, 1, cps1)
        return carry

    lax.fori_loop(0, nch // 2, body, 0)
    if nch % 2:
        finish(nch - 1, 0, start(nch - 1, 0))


@functools.cache
def _build_sc_scatter():
    return functools.partial(
        pl.kernel,
        out_type=(jax.ShapeDtypeStruct((N, D), _f32),
                  jax.ShapeDtypeStruct((N, D), _f32)),
        mesh=_sc_mesh(),
        scratch_types=[
            pltpu.VMEM_SHARED((ACC_ROWS, D), _f32),
            pltpu.VMEM((16, D), _f32),
            pltpu.VMEM((8, CH), jnp.int32),
            pltpu.VMEM((CH, D), _f32),
        ],
    )(_sc_scatter_body)


def _sc_scatter(sig, msg, col):
    return _build_sc_scatter()(sig, msg, col)


def _sc_scatter_body(sig_hbm, msg_hbm, col_hbm, nag_hbm, eag_hbm,
                     acc, zb, lidx, dbuf):
    c = lax.axis_index("c")
    s = lax.axis_index("s")

    zv = jnp.zeros((16,), _f32)
    for i in range(16):
        for g in range(D // 16):
            zb[i, pl.ds(g * 16, 16)] = zv

    def zbody(k, carry):
        pltpu.sync_copy(zb, acc.at[pl.ds(s * ROWS_PER_TILE + k * 16, 16)])
        return carry

    lax.fori_loop(0, ROWS_PER_TILE // 16, zbody, 0)
    plsc.subcore_barrier()

    epw = E // NS

    def make_loop(data_hbm):
        def body(k, carry):
            eb = s * epw + k * CH
            pltpu.sync_copy(col_hbm.at[pl.ds(eb, CH)], lidx.at[0])
            pltpu.sync_copy(data_hbm.at[pl.ds(eb, CH)], dbuf)
            pltpu.sync_copy(dbuf, acc.at[lidx.at[0]], add=True)
            return carry
        return body

    @pl.when(c == 0)
    def _():
        lax.fori_loop(0, epw // CH, make_loop(sig_hbm), 0)

    @pl.when(c == 1)
    def _():
        lax.fori_loop(0, epw // CH, make_loop(msg_hbm), 0)

    plsc.subcore_barrier()

    tail = N - (NS - 1) * ROWS_PER_TILE

    def dump(out_hbm):
        off = s * ROWS_PER_TILE

        @pl.when(s < NS - 1)
        def _():
            pltpu.sync_copy(acc.at[pl.ds(off, ROWS_PER_TILE)],
                            out_hbm.at[pl.ds(off, ROWS_PER_TILE)])

        @pl.when(s == NS - 1)
        def _():
            pltpu.sync_copy(acc.at[pl.ds(off, tail)],
                            out_hbm.at[pl.ds(off, tail)])

    @pl.when(c == 0)
    def _():
        dump(eag_hbm)

    @pl.when(c == 1)
    def _():
        dump(nag_hbm)


def kernel(node_feats, edge_attr, edge_index, Wsg, bsg, Wdg, bdg, Weg, beg,
           g1, b1, Wsu, bsu, Wdu, bdu, g2, b2):
    del bsg, bdg, beg, bsu
    row = edge_index[0]
    col = edge_index[1]
    p_tab = jnp.concatenate([bdu[None], jnp.zeros((7, D), _f32)], 0)
    p_e = jnp.concatenate([g1[None], b1[None], jnp.zeros((6, D), _f32)], 0)
    p_n = jnp.concatenate([g2[None], b2[None], jnp.zeros((6, D), _f32)], 0)

    nb = N // BN_BLK
    eb = E // BE_BLK

    gtab, adg, asu = pl.pallas_call(
        _tables_body,
        grid=(nb,),
        in_specs=[
            pl.BlockSpec((BN_BLK, D), lambda i: (i, 0)),
            pl.BlockSpec((D, D), lambda i: (0, 0)),
            pl.BlockSpec((D, D), lambda i: (0, 0)),
            pl.BlockSpec((D, D), lambda i: (0, 0)),
            pl.BlockSpec((D, D), lambda i: (0, 0)),
            pl.BlockSpec((8, D), lambda i: (0, 0)),
        ],
        out_specs=[
            pl.BlockSpec((BN_BLK, 2 * D), lambda i: (i, 0)),
            pl.BlockSpec((BN_BLK, D), lambda i: (i, 0)),
            pl.BlockSpec((BN_BLK, D), lambda i: (i, 0)),
        ],
        out_shape=[
            jax.ShapeDtypeStruct((N, 2 * D), _f32),
            jax.ShapeDtypeStruct((N, D), _f32),
            jax.ShapeDtypeStruct((N, D), _f32),
        ],
    )(node_feats, Wsg, Wdu, Wdg, Wsu, p_tab)

    gau, gb = _sc_gather(gtab, adg, row, col)

    pre_e, st1 = pl.pallas_call(
        _edge_pre_body,
        grid=(eb,),
        in_specs=[
            pl.BlockSpec((BE_BLK, D), lambda i: (i, 0)),
            pl.BlockSpec((BE_BLK, D), lambda i: (i, 0)),
            pl.BlockSpec((BE_BLK, D), lambda i: (i, 0)),
            pl.BlockSpec((D, D), lambda i: (0, 0)),
        ],
        out_specs=[
            pl.BlockSpec((BE_BLK, D), lambda i: (i, 0)),
            pl.BlockSpec((8, D), lambda i: (0, 0)),
        ],
        out_shape=[
            jax.ShapeDtypeStruct((E, D), _f32),
            jax.ShapeDtypeStruct((8, D), _f32),
        ],
    )(edge_attr, gau, gb, Weg)

    nea, sig, msg = pl.pallas_call(
        _edge_fin_body,
        grid=(eb,),
        in_specs=[
            pl.BlockSpec((BE_BLK, D), lambda i: (i, 0)),
            pl.BlockSpec((BE_BLK, D), lambda i: (i, 0)),
            pl.BlockSpec((BE_BLK, D), lambda i: (i, 1)),
            pl.BlockSpec((8, D), lambda i: (0, 0)),
            pl.BlockSpec((8, D), lambda i: (0, 0)),
        ],
        out_specs=[
            pl.BlockSpec((BE_BLK, D), lambda i: (i, 0)),
            pl.BlockSpec((BE_BLK, D), lambda i: (i, 0)),
            pl.BlockSpec((BE_BLK, D), lambda i: (i, 0)),
        ],
        out_shape=[
            jax.ShapeDtypeStruct((E, D), _f32),
            jax.ShapeDtypeStruct((E, D), _f32),
            jax.ShapeDtypeStruct((E, D), _f32),
        ],
    )(pre_e, edge_attr, gau, st1, p_e)

    nagg, eagg = _sc_scatter(sig, msg, col)

    pre_n, st2 = pl.pallas_call(
        _node_pre_body,
        grid=(nb,),
        in_specs=[
            pl.BlockSpec((BN_BLK, D), lambda i: (i, 0)),
            pl.BlockSpec((BN_BLK, D), lambda i: (i, 0)),
            pl.BlockSpec((BN_BLK, D), lambda i: (i, 0)),
        ],
        out_specs=[
            pl.BlockSpec((BN_BLK, D), lambda i: (i, 0)),
            pl.BlockSpec((8, D), lambda i: (0, 0)),
        ],
        out_shape=[
            jax.ShapeDtypeStruct((N, D), _f32),
            jax.ShapeDtypeStruct((8, D), _f32),
        ],
    )(asu, nagg, eagg)

    new_node_feats = pl.pallas_call(
        _node_fin_body,
        grid=(nb,),
        in_specs=[
            pl.BlockSpec((BN_BLK, D), lambda i: (i, 0)),
            pl.BlockSpec((BN_BLK, D), lambda i: (i, 0)),
            pl.BlockSpec((8, D), lambda i: (0, 0)),
            pl.BlockSpec((8, D), lambda i: (0, 0)),
        ],
        out_specs=pl.BlockSpec((BN_BLK, D), lambda i: (i, 0)),
        out_shape=jax.ShapeDtypeStruct((N, D), _f32),
    )(node_feats, pre_n, st2, p_n)

    return (new_node_feats, nea)


# speedup vs baseline: 2.6527x; 1.0030x over previous
import functools

import jax
import jax.numpy as jnp
from jax import lax
from jax.experimental import pallas as pl
from jax.experimental.pallas import tpu as pltpu
from jax.experimental.pallas import tpu_sc as plsc

N = 10000
E = 320000
D = 128

NC = 2
NS = 16
NW = NC * NS
EH = E // 2
CHG = 40
CH = 80
EPW = EH // NW
ACC_ROWS = 10240
ROWS_PER_TILE = ACC_ROWS // NS

BN_BLK = 400
BE_BLK = 1600

_f32 = jnp.float32
_bf16 = jnp.bfloat16


def _dotT(x, w):
    return lax.dot_general(x, w, (((1,), (1,)), ((), ())),
                           preferred_element_type=_f32)


def _tables_body(x_ref, wsg_ref, wdu_ref, wdg_ref, wsu_ref, p_ref,
                 gau_ref, adg_ref, asu_ref):
    x = x_ref[...]
    gau_ref[:, :D] = _dotT(x, wsg_ref[...])
    gau_ref[:, D:] = _dotT(x, wdu_ref[...]) + p_ref[0][None, :]
    adg_ref[...] = _dotT(x, wdg_ref[...])
    asu_ref[...] = _dotT(x, wsu_ref[...])


def _edge_pre_body(ea_ref, ga_ref, gb_ref, weg_ref, pre_ref, st_ref):
    i = pl.program_id(0)
    pre = _dotT(ea_ref[...], weg_ref[...]) + ga_ref[...] + gb_ref[...]
    pre_ref[...] = pre
    s1 = jnp.sum(pre, axis=0)
    s2 = jnp.sum(pre * pre, axis=0)
    blk = jnp.concatenate([s1[None], s2[None], jnp.zeros((6, D), _f32)], 0)

    @pl.when(i == 0)
    def _():
        st_ref[...] = blk

    @pl.when(i > 0)
    def _():
        st_ref[...] += blk


def _edge_fin_body(pre_ref, ea_ref, au_ref, sta_ref, stb_ref, p_ref,
                   nea_ref, sig_ref, msg_ref):
    st = sta_ref[...] + stb_ref[...]
    mean = st[0] / E
    var = st[1] / E - mean * mean
    inv = lax.rsqrt(var + 1e-5)
    xh = (pre_ref[...] - mean[None, :]) * inv[None, :] * p_ref[0][None, :] \
        + p_ref[1][None, :]
    nea = ea_ref[...] + xh * jax.nn.sigmoid(xh)
    sig = jax.nn.sigmoid(nea)
    nea_ref[...] = nea
    sig_ref[...] = sig
    msg_ref[...] = sig * au_ref[...]


def _edge_fin_body_b(pre_ref, ea_ref, au_ref, sta_ref, stb_ref, p_ref,
                     nea_in_ref, nea_ref, sig_ref, msg_ref):
    del nea_in_ref
    _edge_fin_body(pre_ref, ea_ref, au_ref, sta_ref, stb_ref, p_ref,
                   nea_ref, sig_ref, msg_ref)


def _node_pre_body(asu_ref, na_ref, nb_ref, ea_ref, eb_ref, pre_ref, st_ref):
    i = pl.program_id(0)
    pre = asu_ref[...] + (na_ref[...] + nb_ref[...]) \
        / (ea_ref[...] + eb_ref[...] + 1e-6)
    pre_ref[...] = pre
    s1 = jnp.sum(pre, axis=0)
    s2 = jnp.sum(pre * pre, axis=0)
    blk = jnp.concatenate([s1[None], s2[None], jnp.zeros((6, D), _f32)], 0)

    @pl.when(i == 0)
    def _():
        st_ref[...] = blk

    @pl.when(i > 0)
    def _():
        st_ref[...] += blk


def _node_fin_body(x_ref, pre_ref, st_ref, p_ref, out_ref):
    st = st_ref[...]
    mean = st[0] / N
    var = st[1] / N - mean * mean
    inv = lax.rsqrt(var + 1e-5)
    xh = (pre_ref[...] - mean[None, :]) * inv[None, :] * p_ref[0][None, :] \
        + p_ref[1][None, :]
    out_ref[...] = x_ref[...] + xh * jax.nn.sigmoid(xh)


def _sc_mesh():
    return plsc.VectorSubcoreMesh(core_axis_name="c", subcore_axis_name="s",
                                  num_cores=NC, num_subcores=NS)


@functools.cache
def _build_sc_gather(ebase):
    return functools.partial(
        pl.kernel,
        out_type=(jax.ShapeDtypeStruct((EH, 2 * D), _f32),
                  jax.ShapeDtypeStruct((EH, D), _f32)),
        mesh=_sc_mesh(),
        scratch_types=[
            pltpu.VMEM((2, CHG), jnp.int32),
            pltpu.VMEM((2, CHG), jnp.int32),
            pltpu.VMEM((2, CHG, 2 * D), _f32),
            pltpu.VMEM((2, CHG, D), _f32),
            pltpu.SemaphoreType.DMA,
            pltpu.SemaphoreType.DMA,
            pltpu.SemaphoreType.DMA,
            pltpu.SemaphoreType.DMA,
        ],
    )(functools.partial(_sc_gather_body, ebase))


def _sc_gather(ebase, gtab, adg, row, col):
    return _build_sc_gather(ebase)(gtab, adg, row, col)


def _sc_gather_body(ebase, gtab_hbm, adg_hbm, row_hbm, col_hbm,
                    gau_hbm, gb_hbm, idr, idc, abuf, bbuf, *sems):
    wid = lax.axis_index("s") * NC + lax.axis_index("c")
    nch = EPW // CHG

    def start(k, slot):
        base = wid * EPW + k * CHG
        pltpu.sync_copy(row_hbm.at[pl.ds(ebase + base, CHG)], idr.at[slot])
        pltpu.sync_copy(col_hbm.at[pl.ds(ebase + base, CHG)], idc.at[slot])
        cp1 = pltpu.async_copy(gtab_hbm.at[idr.at[slot]], abuf.at[slot],
                               sems[2 * slot + 0])
        cp2 = pltpu.async_copy(adg_hbm.at[idc.at[slot]], bbuf.at[slot],
                               sems[2 * slot + 1])
        return cp1, cp2

    def finish(k, slot, cps):
        base = wid * EPW + k * CHG
        cps[0].wait()
        pltpu.sync_copy(abuf.at[slot], gau_hbm.at[pl.ds(base, CHG)])
        cps[1].wait()
        pltpu.sync_copy(bbuf.at[slot], gb_hbm.at[pl.ds(base, CHG)])

    def body(j, carry):
        k = j * 2
        cps0 = start(k, 0)
        cps1 = start(k + 1, 1)
        finish(k, 0, cps0)
        finish(k + 1, 1, cps1)
        return carry

    lax.fori_loop(0, nch // 2, body, 0)
    if nch % 2:
        finish(nch - 1, 0, start(nch - 1, 0))


@functools.cache
def _build_sc_scatter(ebase):
    return functools.partial(
        pl.kernel,
        out_type=(jax.ShapeDtypeStruct((N, D), _f32),
                  jax.ShapeDtypeStruct((N, D), _f32)),
        mesh=_sc_mesh(),
        scratch_types=[
            pltpu.VMEM_SHARED((ACC_ROWS, D), _f32),
            pltpu.VMEM((16, D), _f32),
            pltpu.VMEM((8, CH), jnp.int32),
            pltpu.VMEM((CH, D), _f32),
        ],
    )(functools.partial(_sc_scatter_body, ebase))


def _sc_scatter(ebase, sig, msg, col):
    return _build_sc_scatter(ebase)(sig, msg, col)


def _sc_scatter_body(ebase, sig_hbm, msg_hbm, col_hbm, nag_hbm, eag_hbm,
                     acc, zb, lidx, dbuf):
    c = lax.axis_index("c")
    s = lax.axis_index("s")

    zv = jnp.zeros((16,), _f32)
    for i in range(16):
        for g in range(D // 16):
            zb[i, pl.ds(g * 16, 16)] = zv

    def zbody(k, carry):
        pltpu.sync_copy(zb, acc.at[pl.ds(s * ROWS_PER_TILE + k * 16, 16)])
        return carry

    lax.fori_loop(0, ROWS_PER_TILE // 16, zbody, 0)
    plsc.subcore_barrier()

    epw = EH // NS

    def make_loop(data_hbm):
        def body(k, carry):
            eb = s * epw + k * CH
            pltpu.sync_copy(col_hbm.at[pl.ds(ebase + eb, CH)], lidx.at[0])
            pltpu.sync_copy(data_hbm.at[pl.ds(eb, CH)], dbuf)
            pltpu.sync_copy(dbuf, acc.at[lidx.at[0]], add=True)
            return carry
        return body

    @pl.when(c == 0)
    def _():
        lax.fori_loop(0, epw // CH, make_loop(sig_hbm), 0)

    @pl.when(c == 1)
    def _():
        lax.fori_loop(0, epw // CH, make_loop(msg_hbm), 0)

    plsc.subcore_barrier()

    tail = N - (NS - 1) * ROWS_PER_TILE

    def dump(out_hbm):
        off = s * ROWS_PER_TILE

        @pl.when(s < NS - 1)
        def _():
            pltpu.sync_copy(acc.at[pl.ds(off, ROWS_PER_TILE)],
                            out_hbm.at[pl.ds(off, ROWS_PER_TILE)])

        @pl.when(s == NS - 1)
        def _():
            pltpu.sync_copy(acc.at[pl.ds(off, tail)],
                            out_hbm.at[pl.ds(off, tail)])

    @pl.when(c == 0)
    def _():
        dump(eag_hbm)

    @pl.when(c == 1)
    def _():
        dump(nag_hbm)


def kernel(node_feats, edge_attr, edge_index, Wsg, bsg, Wdg, bdg, Weg, beg,
           g1, b1, Wsu, bsu, Wdu, bdu, g2, b2):
    del bsg, bdg, beg, bsu
    row = edge_index[0]
    col = edge_index[1]
    p_tab = jnp.concatenate([bdu[None], jnp.zeros((7, D), _f32)], 0)
    p_e = jnp.concatenate([g1[None], b1[None], jnp.zeros((6, D), _f32)], 0)
    p_n = jnp.concatenate([g2[None], b2[None], jnp.zeros((6, D), _f32)], 0)

    nb = N // BN_BLK
    eb = E // BE_BLK

    gtab, adg, asu = pl.pallas_call(
        _tables_body,
        grid=(nb,),
        in_specs=[
            pl.BlockSpec((BN_BLK, D), lambda i: (i, 0)),
            pl.BlockSpec((D, D), lambda i: (0, 0)),
            pl.BlockSpec((D, D), lambda i: (0, 0)),
            pl.BlockSpec((D, D), lambda i: (0, 0)),
            pl.BlockSpec((D, D), lambda i: (0, 0)),
            pl.BlockSpec((8, D), lambda i: (0, 0)),
        ],
        out_specs=[
            pl.BlockSpec((BN_BLK, 2 * D), lambda i: (i, 0)),
            pl.BlockSpec((BN_BLK, D), lambda i: (i, 0)),
            pl.BlockSpec((BN_BLK, D), lambda i: (i, 0)),
        ],
        out_shape=[
            jax.ShapeDtypeStruct((N, 2 * D), _f32),
            jax.ShapeDtypeStruct((N, D), _f32),
            jax.ShapeDtypeStruct((N, D), _f32),
        ],
    )(node_feats, Wsg, Wdu, Wdg, Wsu, p_tab)

    gau1, gb1 = _sc_gather(0, gtab, adg, row, col)
    gau2, gb2 = _sc_gather(EH, gtab, adg, row, col)

    ebh = EH // BE_BLK

    def _tc2(gau, gb, base_blk):
        return pl.pallas_call(
            _edge_pre_body,
            grid=(ebh,),
            in_specs=[
                pl.BlockSpec((BE_BLK, D), lambda i, b=base_blk: (i + b, 0)),
                pl.BlockSpec((BE_BLK, D), lambda i: (i, 0)),
                pl.BlockSpec((BE_BLK, D), lambda i: (i, 0)),
                pl.BlockSpec((D, D), lambda i: (0, 0)),
            ],
            out_specs=[
                pl.BlockSpec((BE_BLK, D), lambda i: (i, 0)),
                pl.BlockSpec((8, D), lambda i: (0, 0)),
            ],
            out_shape=[
                jax.ShapeDtypeStruct((EH, D), _f32),
                jax.ShapeDtypeStruct((8, D), _f32),
            ],
        )(edge_attr, gau, gb, Weg)

    pre1, st1a = _tc2(gau1, gb1, 0)
    pre2, st1b = _tc2(gau2, gb2, ebh)

    _sml = pl.BlockSpec((8, D), lambda i: (0, 0))

    nea1, sig1, msg1 = pl.pallas_call(
        _edge_fin_body,
        grid=(ebh,),
        in_specs=[
            pl.BlockSpec((BE_BLK, D), lambda i: (i, 0)),
            pl.BlockSpec((BE_BLK, D), lambda i: (i, 0)),
            pl.BlockSpec((BE_BLK, D), lambda i: (i, 1)),
            _sml, _sml, _sml,
        ],
        out_specs=[
            pl.BlockSpec((BE_BLK, D), lambda i: (i, 0)),
            pl.BlockSpec((BE_BLK, D), lambda i: (i, 0)),
            pl.BlockSpec((BE_BLK, D), lambda i: (i, 0)),
        ],
        out_shape=[
            jax.ShapeDtypeStruct((E, D), _f32),
            jax.ShapeDtypeStruct((EH, D), _f32),
            jax.ShapeDtypeStruct((EH, D), _f32),
        ],
    )(pre1, edge_attr, gau1, st1a, st1b, p_e)

    nag1, eag1 = _sc_scatter(0, sig1, msg1, col)

    nea, sig2, msg2 = pl.pallas_call(
        _edge_fin_body_b,
        grid=(ebh,),
        in_specs=[
            pl.BlockSpec((BE_BLK, D), lambda i: (i, 0)),
            pl.BlockSpec((BE_BLK, D), lambda i, b=ebh: (i + b, 0)),
            pl.BlockSpec((BE_BLK, D), lambda i: (i, 1)),
            _sml, _sml, _sml,
            pl.BlockSpec(memory_space=pl.ANY),
        ],
        out_specs=[
            pl.BlockSpec((BE_BLK, D), lambda i, b=ebh: (i + b, 0)),
            pl.BlockSpec((BE_BLK, D), lambda i: (i, 0)),
            pl.BlockSpec((BE_BLK, D), lambda i: (i, 0)),
        ],
        out_shape=[
            jax.ShapeDtypeStruct((E, D), _f32),
            jax.ShapeDtypeStruct((EH, D), _f32),
            jax.ShapeDtypeStruct((EH, D), _f32),
        ],
        input_output_aliases={6: 0},
    )(pre2, edge_attr, gau2, st1a, st1b, p_e, nea1)

    nag2, eag2 = _sc_scatter(EH, sig2, msg2, col)

    pre_n, st2 = pl.pallas_call(
        _node_pre_body,
        grid=(nb,),
        in_specs=[
            pl.BlockSpec((BN_BLK, D), lambda i: (i, 0)),
            pl.BlockSpec((BN_BLK, D), lambda i: (i, 0)),
            pl.BlockSpec((BN_BLK, D), lambda i: (i, 0)),
            pl.BlockSpec((BN_BLK, D), lambda i: (i, 0)),
            pl.BlockSpec((BN_BLK, D), lambda i: (i, 0)),
        ],
        out_specs=[
            pl.BlockSpec((BN_BLK, D), lambda i: (i, 0)),
            pl.BlockSpec((8, D), lambda i: (0, 0)),
        ],
        out_shape=[
            jax.ShapeDtypeStruct((N, D), _f32),
            jax.ShapeDtypeStruct((8, D), _f32),
        ],
    )(asu, nag1, nag2, eag1, eag2)

    new_node_feats = pl.pallas_call(
        _node_fin_body,
        grid=(nb,),
        in_specs=[
            pl.BlockSpec((BN_BLK, D), lambda i: (i, 0)),
            pl.BlockSpec((BN_BLK, D), lambda i: (i, 0)),
            pl.BlockSpec((8, D), lambda i: (0, 0)),
            pl.BlockSpec((8, D), lambda i: (0, 0)),
        ],
        out_specs=pl.BlockSpec((BN_BLK, D), lambda i: (i, 0)),
        out_shape=jax.ShapeDtypeStruct((N, D), _f32),
    )(node_feats, pre_n, st2, p_n)

    return (new_node_feats, nea)


# speedup vs baseline: 2.8199x; 1.0630x over previous
import functools

import jax
import jax.numpy as jnp
from jax import lax
from jax.experimental import pallas as pl
from jax.experimental.pallas import tpu as pltpu
from jax.experimental.pallas import tpu_sc as plsc

N = 10000
E = 320000
D = 128

NC = 2
NS = 16
NW = NC * NS
EH1 = 163840
EH2 = E - EH1
CH = 80
ACC_ROWS = 10240
ROWS_PER_TILE = ACC_ROWS // NS

BN_BLK = 400
BE_BLK = 1280

_f32 = jnp.float32
_bf16 = jnp.bfloat16


def _dotT(x, w):
    return lax.dot_general(x, w, (((1,), (1,)), ((), ())),
                           preferred_element_type=_f32)


def _tables_body(x_ref, wsg_ref, wdu_ref, wdg_ref, wsu_ref, p_ref,
                 gau_ref, adg_ref, asu_ref):
    x = x_ref[...]
    gau_ref[:, :D] = _dotT(x, wsg_ref[...])
    gau_ref[:, D:] = _dotT(x, wdu_ref[...]) + p_ref[0][None, :]
    adg_ref[...] = _dotT(x, wdg_ref[...])
    asu_ref[...] = _dotT(x, wsu_ref[...])


def _edge_pre_body(ea_ref, ga_ref, gb_ref, weg_ref, pre_ref, st_ref):
    i = pl.program_id(0)
    pre = _dotT(ea_ref[...], weg_ref[...]) + ga_ref[...] + gb_ref[...]
    pre_ref[...] = pre
    s1 = jnp.sum(pre, axis=0)
    s2 = jnp.sum(pre * pre, axis=0)
    blk = jnp.concatenate([s1[None], s2[None], jnp.zeros((6, D), _f32)], 0)

    @pl.when(i == 0)
    def _():
        st_ref[...] = blk

    @pl.when(i > 0)
    def _():
        st_ref[...] += blk


def _edge_fin_body(pre_ref, ea_ref, au_ref, sta_ref, stb_ref, p_ref,
                   nea_ref, sig_ref, msg_ref):
    st = sta_ref[...] + stb_ref[...]
    mean = st[0] / E
    var = st[1] / E - mean * mean
    inv = lax.rsqrt(var + 1e-5)
    xh = (pre_ref[...] - mean[None, :]) * inv[None, :] * p_ref[0][None, :] \
        + p_ref[1][None, :]
    nea = ea_ref[...] + xh * jax.nn.sigmoid(xh)
    sig = jax.nn.sigmoid(nea)
    nea_ref[...] = nea
    sig_ref[...] = sig
    msg_ref[...] = sig * au_ref[...]


def _edge_fin_body_b(pre_ref, ea_ref, au_ref, sta_ref, stb_ref, p_ref,
                     nea_in_ref, nea_ref, sig_ref, msg_ref):
    del nea_in_ref
    _edge_fin_body(pre_ref, ea_ref, au_ref, sta_ref, stb_ref, p_ref,
                   nea_ref, sig_ref, msg_ref)


def _node_pre_body(asu_ref, na_ref, nb_ref, ea_ref, eb_ref, pre_ref, st_ref):
    i = pl.program_id(0)
    pre = asu_ref[...] + (na_ref[...] + nb_ref[...]) \
        / (ea_ref[...] + eb_ref[...] + 1e-6)
    pre_ref[...] = pre
    s1 = jnp.sum(pre, axis=0)
    s2 = jnp.sum(pre * pre, axis=0)
    blk = jnp.concatenate([s1[None], s2[None], jnp.zeros((6, D), _f32)], 0)

    @pl.when(i == 0)
    def _():
        st_ref[...] = blk

    @pl.when(i > 0)
    def _():
        st_ref[...] += blk


def _node_fin_body(x_ref, pre_ref, st_ref, p_ref, out_ref):
    st = st_ref[...]
    mean = st[0] / N
    var = st[1] / N - mean * mean
    inv = lax.rsqrt(var + 1e-5)
    xh = (pre_ref[...] - mean[None, :]) * inv[None, :] * p_ref[0][None, :] \
        + p_ref[1][None, :]
    out_ref[...] = x_ref[...] + xh * jax.nn.sigmoid(xh)


def _sc_mesh():
    return plsc.VectorSubcoreMesh(core_axis_name="c", subcore_axis_name="s",
                                  num_cores=NC, num_subcores=NS)


@functools.cache
def _build_sc_gather(ebase, ecount):
    return functools.partial(
        pl.kernel,
        out_type=(jax.ShapeDtypeStruct((ecount, 2 * D), _f32),
                  jax.ShapeDtypeStruct((ecount, D), _f32)),
        mesh=_sc_mesh(),
        scratch_types=[
            pltpu.VMEM((2, CH), jnp.int32),
            pltpu.VMEM((2, CH), jnp.int32),
            pltpu.VMEM((2, CH, 2 * D), _f32),
            pltpu.VMEM((2, CH, D), _f32),
            pltpu.SemaphoreType.DMA,
            pltpu.SemaphoreType.DMA,
            pltpu.SemaphoreType.DMA,
            pltpu.SemaphoreType.DMA,
        ],
    )(functools.partial(_sc_gather_body, ebase, ecount))


def _sc_gather(ebase, ecount, gtab, adg, row, col):
    return _build_sc_gather(ebase, ecount)(gtab, adg, row, col)


def _sc_gather_body(ebase, ecount, gtab_hbm, adg_hbm, row_hbm, col_hbm,
                    gau_hbm, gb_hbm, idr, idc, abuf, bbuf, *sems):
    wid = lax.axis_index("s") * NC + lax.axis_index("c")
    epw = ecount // NW
    nch = epw // CH

    def start(k, slot):
        base = wid * epw + k * CH
        pltpu.sync_copy(row_hbm.at[pl.ds(ebase + base, CH)], idr.at[slot])
        pltpu.sync_copy(col_hbm.at[pl.ds(ebase + base, CH)], idc.at[slot])
        cp1 = pltpu.async_copy(gtab_hbm.at[idr.at[slot]], abuf.at[slot],
                               sems[2 * slot + 0])
        cp2 = pltpu.async_copy(adg_hbm.at[idc.at[slot]], bbuf.at[slot],
                               sems[2 * slot + 1])
        return cp1, cp2

    def finish(k, slot, cps):
        base = wid * epw + k * CH
        cps[0].wait()
        pltpu.sync_copy(abuf.at[slot], gau_hbm.at[pl.ds(base, CH)])
        cps[1].wait()
        pltpu.sync_copy(bbuf.at[slot], gb_hbm.at[pl.ds(base, CH)])

    def body(j, carry):
        k = j * 2
        cps0 = start(k, 0)
        cps1 = start(k + 1, 1)
        finish(k, 0, cps0)
        finish(k + 1, 1, cps1)
        return carry

    lax.fori_loop(0, nch // 2, body, 0)
    if nch % 2:
        finish(nch - 1, 0, start(nch - 1, 0))


@functools.cache
def _build_sc_scatter(ebase, ecount):
    return functools.partial(
        pl.kernel,
        out_type=(jax.ShapeDtypeStruct((N, D), _f32),
                  jax.ShapeDtypeStruct((N, D), _f32)),
        mesh=_sc_mesh(),
        scratch_types=[
            pltpu.VMEM_SHARED((ACC_ROWS, D), _f32),
            pltpu.VMEM((64, D), _f32),
            pltpu.VMEM((8, CH), jnp.int32),
            pltpu.VMEM((CH, D), _f32),
        ],
    )(functools.partial(_sc_scatter_body, ebase, ecount))


def _sc_scatter(ebase, ecount, sig, msg, col):
    return _build_sc_scatter(ebase, ecount)(sig, msg, col)


def _sc_scatter_body(ebase, ecount, sig_hbm, msg_hbm, col_hbm,
                     nag_hbm, eag_hbm, acc, zb, lidx, dbuf):
    c = lax.axis_index("c")
    s = lax.axis_index("s")

    zv = jnp.zeros((16,), _f32)
    for i in range(64):
        for g in range(D // 16):
            zb[i, pl.ds(g * 16, 16)] = zv

    def zbody(k, carry):
        pltpu.sync_copy(zb, acc.at[pl.ds(s * ROWS_PER_TILE + k * 64, 64)])
        return carry

    lax.fori_loop(0, ROWS_PER_TILE // 64, zbody, 0)
    plsc.subcore_barrier()

    epw = ecount // NS

    def make_loop(data_hbm):
        def body(k, carry):
            eb = s * epw + k * CH
            pltpu.sync_copy(col_hbm.at[pl.ds(ebase + eb, CH)], lidx.at[0])
            pltpu.sync_copy(data_hbm.at[pl.ds(eb, CH)], dbuf)
            pltpu.sync_copy(dbuf, acc.at[lidx.at[0]], add=True)
            return carry
        return body

    @pl.when(c == 0)
    def _():
        lax.fori_loop(0, epw // CH, make_loop(sig_hbm), 0)

    @pl.when(c == 1)
    def _():
        lax.fori_loop(0, epw // CH, make_loop(msg_hbm), 0)

    plsc.subcore_barrier()

    tail = N - (NS - 1) * ROWS_PER_TILE

    def dump(out_hbm):
        off = s * ROWS_PER_TILE

        @pl.when(s < NS - 1)
        def _():
            pltpu.sync_copy(acc.at[pl.ds(off, ROWS_PER_TILE)],
                            out_hbm.at[pl.ds(off, ROWS_PER_TILE)])

        @pl.when(s == NS - 1)
        def _():
            pltpu.sync_copy(acc.at[pl.ds(off, tail)],
                            out_hbm.at[pl.ds(off, tail)])

    @pl.when(c == 0)
    def _():
        dump(eag_hbm)

    @pl.when(c == 1)
    def _():
        dump(nag_hbm)


def kernel(node_feats, edge_attr, edge_index, Wsg, bsg, Wdg, bdg, Weg, beg,
           g1, b1, Wsu, bsu, Wdu, bdu, g2, b2):
    del bsg, bdg, beg, bsu
    row = edge_index[0]
    col = edge_index[1]
    p_tab = jnp.concatenate([bdu[None], jnp.zeros((7, D), _f32)], 0)
    p_e = jnp.concatenate([g1[None], b1[None], jnp.zeros((6, D), _f32)], 0)
    p_n = jnp.concatenate([g2[None], b2[None], jnp.zeros((6, D), _f32)], 0)

    nb = N // BN_BLK
    eb = E // BE_BLK

    gtab, adg, asu = pl.pallas_call(
        _tables_body,
        grid=(nb,),
        in_specs=[
            pl.BlockSpec((BN_BLK, D), lambda i: (i, 0)),
            pl.BlockSpec((D, D), lambda i: (0, 0)),
            pl.BlockSpec((D, D), lambda i: (0, 0)),
            pl.BlockSpec((D, D), lambda i: (0, 0)),
            pl.BlockSpec((D, D), lambda i: (0, 0)),
            pl.BlockSpec((8, D), lambda i: (0, 0)),
        ],
        out_specs=[
            pl.BlockSpec((BN_BLK, 2 * D), lambda i: (i, 0)),
            pl.BlockSpec((BN_BLK, D), lambda i: (i, 0)),
            pl.BlockSpec((BN_BLK, D), lambda i: (i, 0)),
        ],
        out_shape=[
            jax.ShapeDtypeStruct((N, 2 * D), _f32),
            jax.ShapeDtypeStruct((N, D), _f32),
            jax.ShapeDtypeStruct((N, D), _f32),
        ],
    )(node_feats, Wsg, Wdu, Wdg, Wsu, p_tab)

    gau1, gb1 = _sc_gather(0, EH1, gtab, adg, row, col)
    gau2, gb2 = _sc_gather(EH1, EH2, gtab, adg, row, col)

    eb1 = EH1 // BE_BLK
    eb2 = EH2 // BE_BLK

    def _tc2(gau, gb, base_blk, nblk, ecount):
        return pl.pallas_call(
            _edge_pre_body,
            grid=(nblk,),
            in_specs=[
                pl.BlockSpec((BE_BLK, D), lambda i, b=base_blk: (i + b, 0)),
                pl.BlockSpec((BE_BLK, D), lambda i: (i, 0)),
                pl.BlockSpec((BE_BLK, D), lambda i: (i, 0)),
                pl.BlockSpec((D, D), lambda i: (0, 0)),
            ],
            out_specs=[
                pl.BlockSpec((BE_BLK, D), lambda i: (i, 0)),
                pl.BlockSpec((8, D), lambda i: (0, 0)),
            ],
            out_shape=[
                jax.ShapeDtypeStruct((ecount, D), _f32),
                jax.ShapeDtypeStruct((8, D), _f32),
            ],
        )(edge_attr, gau, gb, Weg)

    pre1, st1a = _tc2(gau1, gb1, 0, eb1, EH1)
    pre2, st1b = _tc2(gau2, gb2, eb1, eb2, EH2)

    _sml = pl.BlockSpec((8, D), lambda i: (0, 0))

    nea1, sig1, msg1 = pl.pallas_call(
        _edge_fin_body,
        grid=(eb1,),
        in_specs=[
            pl.BlockSpec((BE_BLK, D), lambda i: (i, 0)),
            pl.BlockSpec((BE_BLK, D), lambda i: (i, 0)),
            pl.BlockSpec((BE_BLK, D), lambda i: (i, 1)),
            _sml, _sml, _sml,
        ],
        out_specs=[
            pl.BlockSpec((BE_BLK, D), lambda i: (i, 0)),
            pl.BlockSpec((BE_BLK, D), lambda i: (i, 0)),
            pl.BlockSpec((BE_BLK, D), lambda i: (i, 0)),
        ],
        out_shape=[
            jax.ShapeDtypeStruct((E, D), _f32),
            jax.ShapeDtypeStruct((EH1, D), _f32),
            jax.ShapeDtypeStruct((EH1, D), _f32),
        ],
    )(pre1, edge_attr, gau1, st1a, st1b, p_e)

    nag1, eag1 = _sc_scatter(0, EH1, sig1, msg1, col)

    nea, sig2, msg2 = pl.pallas_call(
        _edge_fin_body_b,
        grid=(eb2,),
        in_specs=[
            pl.BlockSpec((BE_BLK, D), lambda i: (i, 0)),
            pl.BlockSpec((BE_BLK, D), lambda i, b=eb1: (i + b, 0)),
            pl.BlockSpec((BE_BLK, D), lambda i: (i, 1)),
            _sml, _sml, _sml,
            pl.BlockSpec(memory_space=pl.ANY),
        ],
        out_specs=[
            pl.BlockSpec((BE_BLK, D), lambda i, b=eb1: (i + b, 0)),
            pl.BlockSpec((BE_BLK, D), lambda i: (i, 0)),
            pl.BlockSpec((BE_BLK, D), lambda i: (i, 0)),
        ],
        out_shape=[
            jax.ShapeDtypeStruct((E, D), _f32),
            jax.ShapeDtypeStruct((EH2, D), _f32),
            jax.ShapeDtypeStruct((EH2, D), _f32),
        ],
        input_output_aliases={6: 0},
    )(pre2, edge_attr, gau2, st1a, st1b, p_e, nea1)

    nag2, eag2 = _sc_scatter(EH1, EH2, sig2, msg2, col)

    pre_n, st2 = pl.pallas_call(
        _node_pre_body,
        grid=(nb,),
        in_specs=[
            pl.BlockSpec((BN_BLK, D), lambda i: (i, 0)),
            pl.BlockSpec((BN_BLK, D), lambda i: (i, 0)),
            pl.BlockSpec((BN_BLK, D), lambda i: (i, 0)),
            pl.BlockSpec((BN_BLK, D), lambda i: (i, 0)),
            pl.BlockSpec((BN_BLK, D), lambda i: (i, 0)),
        ],
        out_specs=[
            pl.BlockSpec((BN_BLK, D), lambda i: (i, 0)),
            pl.BlockSpec((8, D), lambda i: (0, 0)),
        ],
        out_shape=[
            jax.ShapeDtypeStruct((N, D), _f32),
            jax.ShapeDtypeStruct((8, D), _f32),
        ],
    )(asu, nag1, nag2, eag1, eag2)

    new_node_feats = pl.pallas_call(
        _node_fin_body,
        grid=(nb,),
        in_specs=[
            pl.BlockSpec((BN_BLK, D), lambda i: (i, 0)),
            pl.BlockSpec((BN_BLK, D), lambda i: (i, 0)),
            pl.BlockSpec((8, D), lambda i: (0, 0)),
            pl.BlockSpec((8, D), lambda i: (0, 0)),
        ],
        out_specs=pl.BlockSpec((BN_BLK, D), lambda i: (i, 0)),
        out_shape=jax.ShapeDtypeStruct((N, D), _f32),
    )(node_feats, pre_n, st2, p_n)

    return (new_node_feats, nea)


# speedup vs baseline: 3.0263x; 1.0732x over previous
import functools

import jax
import jax.numpy as jnp
from jax import lax
from jax.experimental import pallas as pl
from jax.experimental.pallas import tpu as pltpu
from jax.experimental.pallas import tpu_sc as plsc

N = 10000
E = 320000
D = 128

NC = 2
NS = 16
NW = NC * NS
EH1 = 163840
EH2 = E - EH1
CH = 80
ACC_ROWS = 10240
ROWS_PER_TILE = ACC_ROWS // NS

BN_BLK = 400
BE_BLK = 1280

_f32 = jnp.float32
_bf16 = jnp.bfloat16


def _dotT(x, w):
    return lax.dot_general(x, w, (((1,), (1,)), ((), ())),
                           preferred_element_type=_f32)


_HI_MASK = -65536


def _pack2(lo_f32, hi_f32):
    lb = lax.bitcast_convert_type(lo_f32, jnp.int32) + 0x8000
    hb = lax.bitcast_convert_type(hi_f32, jnp.int32) + 0x8000
    return (hb & _HI_MASK) | ((lb >> 16) & 0xffff)


def _unpack_lo(w):
    return lax.bitcast_convert_type(w << 16, _f32)


def _unpack_hi(w):
    return lax.bitcast_convert_type(w & _HI_MASK, _f32)


def _tables_body(x_ref, wsg_ref, wdu_ref, wdg_ref, wsu_ref, p_ref,
                 gau_ref, adg_ref, asu_ref):
    x = x_ref[...]
    asg = _dotT(x, wsg_ref[...])
    adu = _dotT(x, wdu_ref[...]) + p_ref[0][None, :]
    gau_ref[...] = _pack2(asg, adu)
    adg_ref[...] = _dotT(x, wdg_ref[...])
    asu_ref[...] = _dotT(x, wsu_ref[...])


def _edge_pre_body(ea_ref, ga_ref, gb_ref, weg_ref, pre_ref, st_ref):
    i = pl.program_id(0)
    ga = _unpack_lo(ga_ref[...])
    pre = _dotT(ea_ref[...], weg_ref[...]) + ga + gb_ref[...]
    pre_ref[...] = pre
    s1 = jnp.sum(pre, axis=0)
    s2 = jnp.sum(pre * pre, axis=0)
    blk = jnp.concatenate([s1[None], s2[None], jnp.zeros((6, D), _f32)], 0)

    @pl.when(i == 0)
    def _():
        st_ref[...] = blk

    @pl.when(i > 0)
    def _():
        st_ref[...] += blk


def _edge_fin_body(pre_ref, ea_ref, au_ref, sta_ref, stb_ref, p_ref,
                   nea_ref, sig_ref, msg_ref):
    st = sta_ref[...] + stb_ref[...]
    mean = st[0] / E
    var = st[1] / E - mean * mean
    inv = lax.rsqrt(var + 1e-5)
    xh = (pre_ref[...] - mean[None, :]) * inv[None, :] * p_ref[0][None, :] \
        + p_ref[1][None, :]
    nea = ea_ref[...] + xh * jax.nn.sigmoid(xh)
    sig = jax.nn.sigmoid(nea)
    nea_ref[...] = nea
    sig_ref[...] = sig
    msg_ref[...] = sig * _unpack_hi(au_ref[...])


def _edge_fin_body_b(pre_ref, ea_ref, au_ref, sta_ref, stb_ref, p_ref,
                     nea_in_ref, nea_ref, sig_ref, msg_ref):
    del nea_in_ref
    _edge_fin_body(pre_ref, ea_ref, au_ref, sta_ref, stb_ref, p_ref,
                   nea_ref, sig_ref, msg_ref)


def _node_pre_body(asu_ref, na_ref, nb_ref, ea_ref, eb_ref, pre_ref, st_ref):
    i = pl.program_id(0)
    pre = asu_ref[...] + (na_ref[...] + nb_ref[...]) \
        / (ea_ref[...] + eb_ref[...] + 1e-6)
    pre_ref[...] = pre
    s1 = jnp.sum(pre, axis=0)
    s2 = jnp.sum(pre * pre, axis=0)
    blk = jnp.concatenate([s1[None], s2[None], jnp.zeros((6, D), _f32)], 0)

    @pl.when(i == 0)
    def _():
        st_ref[...] = blk

    @pl.when(i > 0)
    def _():
        st_ref[...] += blk


def _node_fin_body(x_ref, pre_ref, st_ref, p_ref, out_ref):
    st = st_ref[...]
    mean = st[0] / N
    var = st[1] / N - mean * mean
    inv = lax.rsqrt(var + 1e-5)
    xh = (pre_ref[...] - mean[None, :]) * inv[None, :] * p_ref[0][None, :] \
        + p_ref[1][None, :]
    out_ref[...] = x_ref[...] + xh * jax.nn.sigmoid(xh)


def _sc_mesh():
    return plsc.VectorSubcoreMesh(core_axis_name="c", subcore_axis_name="s",
                                  num_cores=NC, num_subcores=NS)


@functools.cache
def _build_sc_gather(ebase, ecount):
    return functools.partial(
        pl.kernel,
        out_type=(jax.ShapeDtypeStruct((ecount, D), jnp.int32),
                  jax.ShapeDtypeStruct((ecount, D), _f32)),
        mesh=_sc_mesh(),
        scratch_types=[
            pltpu.VMEM((2, CH), jnp.int32),
            pltpu.VMEM((2, CH), jnp.int32),
            pltpu.VMEM((2, CH, D), jnp.int32),
            pltpu.VMEM((2, CH, D), _f32),
            pltpu.SemaphoreType.DMA,
            pltpu.SemaphoreType.DMA,
            pltpu.SemaphoreType.DMA,
            pltpu.SemaphoreType.DMA,
        ],
    )(functools.partial(_sc_gather_body, ebase, ecount))


def _sc_gather(ebase, ecount, gtab, adg, row, col):
    return _build_sc_gather(ebase, ecount)(gtab, adg, row, col)


def _sc_gather_body(ebase, ecount, gtab_hbm, adg_hbm, row_hbm, col_hbm,
                    gau_hbm, gb_hbm, idr, idc, abuf, bbuf, *sems):
    wid = lax.axis_index("s") * NC + lax.axis_index("c")
    epw = ecount // NW
    nch = epw // CH

    def start(k, slot):
        base = wid * epw + k * CH
        pltpu.sync_copy(row_hbm.at[pl.ds(ebase + base, CH)], idr.at[slot])
        pltpu.sync_copy(col_hbm.at[pl.ds(ebase + base, CH)], idc.at[slot])
        cp1 = pltpu.async_copy(gtab_hbm.at[idr.at[slot]], abuf.at[slot],
                               sems[2 * slot + 0])
        cp2 = pltpu.async_copy(adg_hbm.at[idc.at[slot]], bbuf.at[slot],
                               sems[2 * slot + 1])
        return cp1, cp2

    def finish(k, slot, cps):
        base = wid * epw + k * CH
        cps[0].wait()
        pltpu.sync_copy(abuf.at[slot], gau_hbm.at[pl.ds(base, CH)])
        cps[1].wait()
        pltpu.sync_copy(bbuf.at[slot], gb_hbm.at[pl.ds(base, CH)])

    def body(j, carry):
        k = j * 2
        cps0 = start(k, 0)
        cps1 = start(k + 1, 1)
        finish(k, 0, cps0)
        finish(k + 1, 1, cps1)
        return carry

    lax.fori_loop(0, nch // 2, body, 0)
    if nch % 2:
        finish(nch - 1, 0, start(nch - 1, 0))


@functools.cache
def _build_sc_scatter(ebase, ecount):
    return functools.partial(
        pl.kernel,
        out_type=(jax.ShapeDtypeStruct((N, D), _f32),
                  jax.ShapeDtypeStruct((N, D), _f32)),
        mesh=_sc_mesh(),
        scratch_types=[
            pltpu.VMEM_SHARED((ACC_ROWS, D), _f32),
            pltpu.VMEM((64, D), _f32),
            pltpu.VMEM((8, CH), jnp.int32),
            pltpu.VMEM((CH, D), _f32),
        ],
    )(functools.partial(_sc_scatter_body, ebase, ecount))


def _sc_scatter(ebase, ecount, sig, msg, col):
    return _build_sc_scatter(ebase, ecount)(sig, msg, col)


def _sc_scatter_body(ebase, ecount, sig_hbm, msg_hbm, col_hbm,
                     nag_hbm, eag_hbm, acc, zb, lidx, dbuf):
    c = lax.axis_index("c")
    s = lax.axis_index("s")

    zv = jnp.zeros((16,), _f32)
    for i in range(64):
        for g in range(D // 16):
            zb[i, pl.ds(g * 16, 16)] = zv

    def zbody(k, carry):
        pltpu.sync_copy(zb, acc.at[pl.ds(s * ROWS_PER_TILE + k * 64, 64)])
        return carry

    lax.fori_loop(0, ROWS_PER_TILE // 64, zbody, 0)
    plsc.subcore_barrier()

    epw = ecount // NS

    def make_loop(data_hbm):
        def body(k, carry):
            eb = s * epw + k * CH
            pltpu.sync_copy(col_hbm.at[pl.ds(ebase + eb, CH)], lidx.at[0])
            pltpu.sync_copy(data_hbm.at[pl.ds(eb, CH)], dbuf)
            pltpu.sync_copy(dbuf, acc.at[lidx.at[0]], add=True)
            return carry
        return body

    @pl.when(c == 0)
    def _():
        lax.fori_loop(0, epw // CH, make_loop(sig_hbm), 0)

    @pl.when(c == 1)
    def _():
        lax.fori_loop(0, epw // CH, make_loop(msg_hbm), 0)

    plsc.subcore_barrier()

    tail = N - (NS - 1) * ROWS_PER_TILE

    def dump(out_hbm):
        off = s * ROWS_PER_TILE

        @pl.when(s < NS - 1)
        def _():
            pltpu.sync_copy(acc.at[pl.ds(off, ROWS_PER_TILE)],
                            out_hbm.at[pl.ds(off, ROWS_PER_TILE)])

        @pl.when(s == NS - 1)
        def _():
            pltpu.sync_copy(acc.at[pl.ds(off, tail)],
                            out_hbm.at[pl.ds(off, tail)])

    @pl.when(c == 0)
    def _():
        dump(eag_hbm)

    @pl.when(c == 1)
    def _():
        dump(nag_hbm)


def kernel(node_feats, edge_attr, edge_index, Wsg, bsg, Wdg, bdg, Weg, beg,
           g1, b1, Wsu, bsu, Wdu, bdu, g2, b2):
    del bsg, bdg, beg, bsu
    row = edge_index[0]
    col = edge_index[1]
    p_tab = jnp.concatenate([bdu[None], jnp.zeros((7, D), _f32)], 0)
    p_e = jnp.concatenate([g1[None], b1[None], jnp.zeros((6, D), _f32)], 0)
    p_n = jnp.concatenate([g2[None], b2[None], jnp.zeros((6, D), _f32)], 0)

    nb = N // BN_BLK
    eb = E // BE_BLK

    gtab, adg, asu = pl.pallas_call(
        _tables_body,
        grid=(nb,),
        in_specs=[
            pl.BlockSpec((BN_BLK, D), lambda i: (i, 0)),
            pl.BlockSpec((D, D), lambda i: (0, 0)),
            pl.BlockSpec((D, D), lambda i: (0, 0)),
            pl.BlockSpec((D, D), lambda i: (0, 0)),
            pl.BlockSpec((D, D), lambda i: (0, 0)),
            pl.BlockSpec((8, D), lambda i: (0, 0)),
        ],
        out_specs=[
            pl.BlockSpec((BN_BLK, D), lambda i: (i, 0)),
            pl.BlockSpec((BN_BLK, D), lambda i: (i, 0)),
            pl.BlockSpec((BN_BLK, D), lambda i: (i, 0)),
        ],
        out_shape=[
            jax.ShapeDtypeStruct((N, D), jnp.int32),
            jax.ShapeDtypeStruct((N, D), _f32),
            jax.ShapeDtypeStruct((N, D), _f32),
        ],
    )(node_feats, Wsg, Wdu, Wdg, Wsu, p_tab)

    gau1, gb1 = _sc_gather(0, EH1, gtab, adg, row, col)
    gau2, gb2 = _sc_gather(EH1, EH2, gtab, adg, row, col)

    eb1 = EH1 // BE_BLK
    eb2 = EH2 // BE_BLK

    def _tc2(gau, gb, base_blk, nblk, ecount):
        return pl.pallas_call(
            _edge_pre_body,
            grid=(nblk,),
            in_specs=[
                pl.BlockSpec((BE_BLK, D), lambda i, b=base_blk: (i + b, 0)),
                pl.BlockSpec((BE_BLK, D), lambda i: (i, 0)),
                pl.BlockSpec((BE_BLK, D), lambda i: (i, 0)),
                pl.BlockSpec((D, D), lambda i: (0, 0)),
            ],
            out_specs=[
                pl.BlockSpec((BE_BLK, D), lambda i: (i, 0)),
                pl.BlockSpec((8, D), lambda i: (0, 0)),
            ],
            out_shape=[
                jax.ShapeDtypeStruct((ecount, D), _f32),
                jax.ShapeDtypeStruct((8, D), _f32),
            ],
        )(edge_attr, gau, gb, Weg)

    pre1, st1a = _tc2(gau1, gb1, 0, eb1, EH1)
    pre2, st1b = _tc2(gau2, gb2, eb1, eb2, EH2)

    _sml = pl.BlockSpec((8, D), lambda i: (0, 0))

    nea1, sig1, msg1 = pl.pallas_call(
        _edge_fin_body,
        grid=(eb1,),
        in_specs=[
            pl.BlockSpec((BE_BLK, D), lambda i: (i, 0)),
            pl.BlockSpec((BE_BLK, D), lambda i: (i, 0)),
            pl.BlockSpec((BE_BLK, D), lambda i: (i, 0)),
            _sml, _sml, _sml,
        ],
        out_specs=[
            pl.BlockSpec((BE_BLK, D), lambda i: (i, 0)),
            pl.BlockSpec((BE_BLK, D), lambda i: (i, 0)),
            pl.BlockSpec((BE_BLK, D), lambda i: (i, 0)),
        ],
        out_shape=[
            jax.ShapeDtypeStruct((E, D), _f32),
            jax.ShapeDtypeStruct((EH1, D), _f32),
            jax.ShapeDtypeStruct((EH1, D), _f32),
        ],
    )(pre1, edge_attr, gau1, st1a, st1b, p_e)

    nag1, eag1 = _sc_scatter(0, EH1, sig1, msg1, col)

    nea, sig2, msg2 = pl.pallas_call(
        _edge_fin_body_b,
        grid=(eb2,),
        in_specs=[
            pl.BlockSpec((BE_BLK, D), lambda i: (i, 0)),
            pl.BlockSpec((BE_BLK, D), lambda i, b=eb1: (i + b, 0)),
            pl.BlockSpec((BE_BLK, D), lambda i: (i, 0)),
            _sml, _sml, _sml,
            pl.BlockSpec(memory_space=pl.ANY),
        ],
        out_specs=[
            pl.BlockSpec((BE_BLK, D), lambda i, b=eb1: (i + b, 0)),
            pl.BlockSpec((BE_BLK, D), lambda i: (i, 0)),
            pl.BlockSpec((BE_BLK, D), lambda i: (i, 0)),
        ],
        out_shape=[
            jax.ShapeDtypeStruct((E, D), _f32),
            jax.ShapeDtypeStruct((EH2, D), _f32),
            jax.ShapeDtypeStruct((EH2, D), _f32),
        ],
        input_output_aliases={6: 0},
    )(pre2, edge_attr, gau2, st1a, st1b, p_e, nea1)

    nag2, eag2 = _sc_scatter(EH1, EH2, sig2, msg2, col)

    pre_n, st2 = pl.pallas_call(
        _node_pre_body,
        grid=(nb,),
        in_specs=[
            pl.BlockSpec((BN_BLK, D), lambda i: (i, 0)),
            pl.BlockSpec((BN_BLK, D), lambda i: (i, 0)),
            pl.BlockSpec((BN_BLK, D), lambda i: (i, 0)),
            pl.BlockSpec((BN_BLK, D), lambda i: (i, 0)),
            pl.BlockSpec((BN_BLK, D), lambda i: (i, 0)),
        ],
        out_specs=[
            pl.BlockSpec((BN_BLK, D), lambda i: (i, 0)),
            pl.BlockSpec((8, D), lambda i: (0, 0)),
        ],
        out_shape=[
            jax.ShapeDtypeStruct((N, D), _f32),
            jax.ShapeDtypeStruct((8, D), _f32),
        ],
    )(asu, nag1, nag2, eag1, eag2)

    new_node_feats = pl.pallas_call(
        _node_fin_body,
        grid=(nb,),
        in_specs=[
            pl.BlockSpec((BN_BLK, D), lambda i: (i, 0)),
            pl.BlockSpec((BN_BLK, D), lambda i: (i, 0)),
            pl.BlockSpec((8, D), lambda i: (0, 0)),
            pl.BlockSpec((8, D), lambda i: (0, 0)),
        ],
        out_specs=pl.BlockSpec((BN_BLK, D), lambda i: (i, 0)),
        out_shape=jax.ShapeDtypeStruct((N, D), _f32),
    )(node_feats, pre_n, st2, p_n)

    return (new_node_feats, nea)


# speedup vs baseline: 3.3603x; 1.1104x over previous
import functools

import jax
import jax.numpy as jnp
from jax import lax
from jax.experimental import pallas as pl
from jax.experimental.pallas import tpu as pltpu
from jax.experimental.pallas import tpu_sc as plsc

N = 10000
E = 320000
D = 128

NC = 2
NS = 16
NW = NC * NS
EH1 = 163840
EH2 = E - EH1
CH = 80
SCH = 160
ACC_ROWS = 10240
ROWS_PER_TILE = ACC_ROWS // NS

BN_BLK = 400
BE_BLK = 1280

_f32 = jnp.float32
_bf16 = jnp.bfloat16


def _dotT(x, w):
    return lax.dot_general(x, w, (((1,), (1,)), ((), ())),
                           preferred_element_type=_f32)


_HI_MASK = -65536


def _pack2(lo_f32, hi_f32):
    lb = lax.bitcast_convert_type(lo_f32, jnp.int32) + 0x8000
    hb = lax.bitcast_convert_type(hi_f32, jnp.int32) + 0x8000
    return (hb & _HI_MASK) | ((lb >> 16) & 0xffff)


def _unpack_lo(w):
    return lax.bitcast_convert_type(w << 16, _f32)


def _unpack_hi(w):
    return lax.bitcast_convert_type(w & _HI_MASK, _f32)


def _tables_body(x_ref, wsg_ref, wdu_ref, wdg_ref, wsu_ref, p_ref,
                 gau_ref, adg_ref, asu_ref):
    x = x_ref[...]
    asg = _dotT(x, wsg_ref[...])
    adu = _dotT(x, wdu_ref[...]) + p_ref[0][None, :]
    gau_ref[...] = _pack2(asg, adu)
    adg_ref[...] = _dotT(x, wdg_ref[...])
    asu_ref[...] = _dotT(x, wsu_ref[...])


def _edge_pre_body(ea_ref, ga_ref, gb_ref, weg_ref, pre_ref, st_ref):
    i = pl.program_id(0)
    ga = _unpack_lo(ga_ref[...])
    pre = _dotT(ea_ref[...], weg_ref[...]) + ga + gb_ref[...]
    pre_ref[...] = pre
    s1 = jnp.sum(pre, axis=0)
    s2 = jnp.sum(pre * pre, axis=0)
    blk = jnp.concatenate([s1[None], s2[None], jnp.zeros((6, D), _f32)], 0)

    @pl.when(i == 0)
    def _():
        st_ref[...] = blk

    @pl.when(i > 0)
    def _():
        st_ref[...] += blk


def _edge_fin_body(pre_ref, ea_ref, au_ref, sta_ref, stb_ref, p_ref,
                   nea_ref, sig_ref, msg_ref):
    st = sta_ref[...] + stb_ref[...]
    mean = st[0] / E
    var = st[1] / E - mean * mean
    inv = lax.rsqrt(var + 1e-5)
    xh = (pre_ref[...] - mean[None, :]) * inv[None, :] * p_ref[0][None, :] \
        + p_ref[1][None, :]
    nea = ea_ref[...] + xh * jax.nn.sigmoid(xh)
    sig = jax.nn.sigmoid(nea)
    nea_ref[...] = nea
    sig_ref[...] = sig
    msg_ref[...] = sig * _unpack_hi(au_ref[...])


def _edge_fin_body_b(pre_ref, ea_ref, au_ref, sta_ref, stb_ref, p_ref,
                     nea_in_ref, nea_ref, sig_ref, msg_ref):
    del nea_in_ref
    _edge_fin_body(pre_ref, ea_ref, au_ref, sta_ref, stb_ref, p_ref,
                   nea_ref, sig_ref, msg_ref)


def _node_pre_body(asu_ref, na_ref, nb_ref, ea_ref, eb_ref, pre_ref, st_ref):
    i = pl.program_id(0)
    pre = asu_ref[...] + (na_ref[...] + nb_ref[...]) \
        / (ea_ref[...] + eb_ref[...] + 1e-6)
    pre_ref[...] = pre
    s1 = jnp.sum(pre, axis=0)
    s2 = jnp.sum(pre * pre, axis=0)
    blk = jnp.concatenate([s1[None], s2[None], jnp.zeros((6, D), _f32)], 0)

    @pl.when(i == 0)
    def _():
        st_ref[...] = blk

    @pl.when(i > 0)
    def _():
        st_ref[...] += blk


def _node_fin_body(x_ref, pre_ref, st_ref, p_ref, out_ref):
    st = st_ref[...]
    mean = st[0] / N
    var = st[1] / N - mean * mean
    inv = lax.rsqrt(var + 1e-5)
    xh = (pre_ref[...] - mean[None, :]) * inv[None, :] * p_ref[0][None, :] \
        + p_ref[1][None, :]
    out_ref[...] = x_ref[...] + xh * jax.nn.sigmoid(xh)


def _sc_mesh():
    return plsc.VectorSubcoreMesh(core_axis_name="c", subcore_axis_name="s",
                                  num_cores=NC, num_subcores=NS)


@functools.cache
def _build_sc_gather(ebase, ecount):
    return functools.partial(
        pl.kernel,
        out_type=(jax.ShapeDtypeStruct((ecount, D), jnp.int32),
                  jax.ShapeDtypeStruct((ecount, D), _f32)),
        mesh=_sc_mesh(),
        scratch_types=[
            pltpu.VMEM((2, CH), jnp.int32),
            pltpu.VMEM((2, CH), jnp.int32),
            pltpu.VMEM((2, CH, D), jnp.int32),
            pltpu.VMEM((2, CH, D), _f32),
            pltpu.SemaphoreType.DMA,
            pltpu.SemaphoreType.DMA,
            pltpu.SemaphoreType.DMA,
            pltpu.SemaphoreType.DMA,
        ],
    )(functools.partial(_sc_gather_body, ebase, ecount))


def _sc_gather(ebase, ecount, gtab, adg, row, col):
    return _build_sc_gather(ebase, ecount)(gtab, adg, row, col)


def _sc_gather_body(ebase, ecount, gtab_hbm, adg_hbm, row_hbm, col_hbm,
                    gau_hbm, gb_hbm, idr, idc, abuf, bbuf, *sems):
    wid = lax.axis_index("s") * NC + lax.axis_index("c")
    epw = ecount // NW
    nch = epw // CH

    def start(k, slot):
        base = wid * epw + k * CH
        pltpu.sync_copy(row_hbm.at[pl.ds(ebase + base, CH)], idr.at[slot])
        pltpu.sync_copy(col_hbm.at[pl.ds(ebase + base, CH)], idc.at[slot])
        cp1 = pltpu.async_copy(gtab_hbm.at[idr.at[slot]], abuf.at[slot],
                               sems[2 * slot + 0])
        cp2 = pltpu.async_copy(adg_hbm.at[idc.at[slot]], bbuf.at[slot],
                               sems[2 * slot + 1])
        return cp1, cp2

    def finish(k, slot, cps):
        base = wid * epw + k * CH
        cps[0].wait()
        pltpu.sync_copy(abuf.at[slot], gau_hbm.at[pl.ds(base, CH)])
        cps[1].wait()
        pltpu.sync_copy(bbuf.at[slot], gb_hbm.at[pl.ds(base, CH)])

    def body(j, carry):
        k = j * 2
        cps0 = start(k, 0)
        cps1 = start(k + 1, 1)
        finish(k, 0, cps0)
        finish(k + 1, 1, cps1)
        return carry

    lax.fori_loop(0, nch // 2, body, 0)
    if nch % 2:
        finish(nch - 1, 0, start(nch - 1, 0))


@functools.cache
def _build_sc_scatter(ebase, ecount):
    return functools.partial(
        pl.kernel,
        out_type=(jax.ShapeDtypeStruct((N, D), _f32),
                  jax.ShapeDtypeStruct((N, D), _f32)),
        mesh=_sc_mesh(),
        scratch_types=[
            pltpu.VMEM_SHARED((ACC_ROWS, D), _f32),
            pltpu.VMEM((64, D), _f32),
            pltpu.VMEM((SCH,), jnp.int32),
            pltpu.VMEM((SCH, D), _f32),
        ],
    )(functools.partial(_sc_scatter_body, ebase, ecount))


def _sc_scatter(ebase, ecount, sig, msg, col):
    return _build_sc_scatter(ebase, ecount)(sig, msg, col)


def _sc_scatter_body(ebase, ecount, sig_hbm, msg_hbm, col_hbm,
                     nag_hbm, eag_hbm, acc, zb, lidx, dbuf):
    c = lax.axis_index("c")
    s = lax.axis_index("s")

    zv = jnp.zeros((16,), _f32)
    for i in range(64):
        for g in range(D // 16):
            zb[i, pl.ds(g * 16, 16)] = zv

    def zbody(k, carry):
        pltpu.sync_copy(zb, acc.at[pl.ds(s * ROWS_PER_TILE + k * 64, 64)])
        return carry

    lax.fori_loop(0, ROWS_PER_TILE // 64, zbody, 0)
    plsc.subcore_barrier()

    epw = ecount // NS

    def make_loop(data_hbm):
        def body(k, carry):
            eb = s * epw + k * SCH
            pltpu.sync_copy(col_hbm.at[pl.ds(ebase + eb, SCH)], lidx)
            pltpu.sync_copy(data_hbm.at[pl.ds(eb, SCH)], dbuf)
            pltpu.sync_copy(dbuf, acc.at[lidx], add=True)
            return carry
        return body

    @pl.when(c == 0)
    def _():
        lax.fori_loop(0, epw // SCH, make_loop(sig_hbm), 0)

    @pl.when(c == 1)
    def _():
        lax.fori_loop(0, epw // SCH, make_loop(msg_hbm), 0)

    plsc.subcore_barrier()

    tail = N - (NS - 1) * ROWS_PER_TILE

    def dump(out_hbm):
        off = s * ROWS_PER_TILE

        @pl.when(s < NS - 1)
        def _():
            pltpu.sync_copy(acc.at[pl.ds(off, ROWS_PER_TILE)],
                            out_hbm.at[pl.ds(off, ROWS_PER_TILE)])

        @pl.when(s == NS - 1)
        def _():
            pltpu.sync_copy(acc.at[pl.ds(off, tail)],
                            out_hbm.at[pl.ds(off, tail)])

    @pl.when(c == 0)
    def _():
        dump(eag_hbm)

    @pl.when(c == 1)
    def _():
        dump(nag_hbm)


def kernel(node_feats, edge_attr, edge_index, Wsg, bsg, Wdg, bdg, Weg, beg,
           g1, b1, Wsu, bsu, Wdu, bdu, g2, b2):
    del bsg, bdg, beg, bsu
    row = edge_index[0]
    col = edge_index[1]
    p_tab = jnp.concatenate([bdu[None], jnp.zeros((7, D), _f32)], 0)
    p_e = jnp.concatenate([g1[None], b1[None], jnp.zeros((6, D), _f32)], 0)
    p_n = jnp.concatenate([g2[None], b2[None], jnp.zeros((6, D), _f32)], 0)

    nb = N // BN_BLK
    eb = E // BE_BLK

    gtab, adg, asu = pl.pallas_call(
        _tables_body,
        grid=(nb,),
        in_specs=[
            pl.BlockSpec((BN_BLK, D), lambda i: (i, 0)),
            pl.BlockSpec((D, D), lambda i: (0, 0)),
            pl.BlockSpec((D, D), lambda i: (0, 0)),
            pl.BlockSpec((D, D), lambda i: (0, 0)),
            pl.BlockSpec((D, D), lambda i: (0, 0)),
            pl.BlockSpec((8, D), lambda i: (0, 0)),
        ],
        out_specs=[
            pl.BlockSpec((BN_BLK, D), lambda i: (i, 0)),
            pl.BlockSpec((BN_BLK, D), lambda i: (i, 0)),
            pl.BlockSpec((BN_BLK, D), lambda i: (i, 0)),
        ],
        out_shape=[
            jax.ShapeDtypeStruct((N, D), jnp.int32),
            jax.ShapeDtypeStruct((N, D), _f32),
            jax.ShapeDtypeStruct((N, D), _f32),
        ],
    )(node_feats, Wsg, Wdu, Wdg, Wsu, p_tab)

    gau1, gb1 = _sc_gather(0, EH1, gtab, adg, row, col)
    gau2, gb2 = _sc_gather(EH1, EH2, gtab, adg, row, col)

    eb1 = EH1 // BE_BLK
    eb2 = EH2 // BE_BLK

    def _tc2(gau, gb, base_blk, nblk, ecount):
        return pl.pallas_call(
            _edge_pre_body,
            grid=(nblk,),
            in_specs=[
                pl.BlockSpec((BE_BLK, D), lambda i, b=base_blk: (i + b, 0)),
                pl.BlockSpec((BE_BLK, D), lambda i: (i, 0)),
                pl.BlockSpec((BE_BLK, D), lambda i: (i, 0)),
                pl.BlockSpec((D, D), lambda i: (0, 0)),
            ],
            out_specs=[
                pl.BlockSpec((BE_BLK, D), lambda i: (i, 0)),
                pl.BlockSpec((8, D), lambda i: (0, 0)),
            ],
            out_shape=[
                jax.ShapeDtypeStruct((ecount, D), _f32),
                jax.ShapeDtypeStruct((8, D), _f32),
            ],
        )(edge_attr, gau, gb, Weg)

    pre1, st1a = _tc2(gau1, gb1, 0, eb1, EH1)
    pre2, st1b = _tc2(gau2, gb2, eb1, eb2, EH2)

    _sml = pl.BlockSpec((8, D), lambda i: (0, 0))

    nea1, sig1, msg1 = pl.pallas_call(
        _edge_fin_body,
        grid=(eb1,),
        in_specs=[
            pl.BlockSpec((BE_BLK, D), lambda i: (i, 0)),
            pl.BlockSpec((BE_BLK, D), lambda i: (i, 0)),
            pl.BlockSpec((BE_BLK, D), lambda i: (i, 0)),
            _sml, _sml, _sml,
        ],
        out_specs=[
            pl.BlockSpec((BE_BLK, D), lambda i: (i, 0)),
            pl.BlockSpec((BE_BLK, D), lambda i: (i, 0)),
            pl.BlockSpec((BE_BLK, D), lambda i: (i, 0)),
        ],
        out_shape=[
            jax.ShapeDtypeStruct((E, D), _f32),
            jax.ShapeDtypeStruct((EH1, D), _f32),
            jax.ShapeDtypeStruct((EH1, D), _f32),
        ],
    )(pre1, edge_attr, gau1, st1a, st1b, p_e)

    nag1, eag1 = _sc_scatter(0, EH1, sig1, msg1, col)

    nea, sig2, msg2 = pl.pallas_call(
        _edge_fin_body_b,
        grid=(eb2,),
        in_specs=[
            pl.BlockSpec((BE_BLK, D), lambda i: (i, 0)),
            pl.BlockSpec((BE_BLK, D), lambda i, b=eb1: (i + b, 0)),
            pl.BlockSpec((BE_BLK, D), lambda i: (i, 0)),
            _sml, _sml, _sml,
            pl.BlockSpec(memory_space=pl.ANY),
        ],
        out_specs=[
            pl.BlockSpec((BE_BLK, D), lambda i, b=eb1: (i + b, 0)),
            pl.BlockSpec((BE_BLK, D), lambda i: (i, 0)),
            pl.BlockSpec((BE_BLK, D), lambda i: (i, 0)),
        ],
        out_shape=[
            jax.ShapeDtypeStruct((E, D), _f32),
            jax.ShapeDtypeStruct((EH2, D), _f32),
            jax.ShapeDtypeStruct((EH2, D), _f32),
        ],
        input_output_aliases={6: 0},
    )(pre2, edge_attr, gau2, st1a, st1b, p_e, nea1)

    nag2, eag2 = _sc_scatter(EH1, EH2, sig2, msg2, col)

    pre_n, st2 = pl.pallas_call(
        _node_pre_body,
        grid=(nb,),
        in_specs=[
            pl.BlockSpec((BN_BLK, D), lambda i: (i, 0)),
            pl.BlockSpec((BN_BLK, D), lambda i: (i, 0)),
            pl.BlockSpec((BN_BLK, D), lambda i: (i, 0)),
            pl.BlockSpec((BN_BLK, D), lambda i: (i, 0)),
            pl.BlockSpec((BN_BLK, D), lambda i: (i, 0)),
        ],
        out_specs=[
            pl.BlockSpec((BN_BLK, D), lambda i: (i, 0)),
            pl.BlockSpec((8, D), lambda i: (0, 0)),
        ],
        out_shape=[
            jax.ShapeDtypeStruct((N, D), _f32),
            jax.ShapeDtypeStruct((8, D), _f32),
        ],
    )(asu, nag1, nag2, eag1, eag2)

    new_node_feats = pl.pallas_call(
        _node_fin_body,
        grid=(nb,),
        in_specs=[
            pl.BlockSpec((BN_BLK, D), lambda i: (i, 0)),
            pl.BlockSpec((BN_BLK, D), lambda i: (i, 0)),
            pl.BlockSpec((8, D), lambda i: (0, 0)),
            pl.BlockSpec((8, D), lambda i: (0, 0)),
        ],
        out_specs=pl.BlockSpec((BN_BLK, D), lambda i: (i, 0)),
        out_shape=jax.ShapeDtypeStruct((N, D), _f32),
    )(node_feats, pre_n, st2, p_n)

    return (new_node_feats, nea)


# speedup vs baseline: 3.4443x; 1.0250x over previous
import functools

import jax
import jax.numpy as jnp
from jax import lax
from jax.experimental import pallas as pl
from jax.experimental.pallas import tpu as pltpu
from jax.experimental.pallas import tpu_sc as plsc

N = 10000
E = 320000
D = 128

NC = 2
NS = 16
NW = NC * NS
EH1 = 163840
EH2 = E - EH1
CH = 80
SCH = 160
ACC_ROWS = 10240
ROWS_PER_TILE = ACC_ROWS // NS

BN_BLK = 400
BE_BLK = 1280

_f32 = jnp.float32
_bf16 = jnp.bfloat16


def _dotT(x, w):
    return lax.dot_general(x, w, (((1,), (1,)), ((), ())),
                           preferred_element_type=_f32)


_HI_MASK = -65536


def _pack2(lo_f32, hi_f32):
    lb = lax.bitcast_convert_type(lo_f32, jnp.int32) + 0x8000
    hb = lax.bitcast_convert_type(hi_f32, jnp.int32) + 0x8000
    return (hb & _HI_MASK) | ((lb >> 16) & 0xffff)


def _unpack_lo(w):
    return lax.bitcast_convert_type(w << 16, _f32)


def _unpack_hi(w):
    return lax.bitcast_convert_type(w & _HI_MASK, _f32)


def _tables_body(x_ref, wsg_ref, wdu_ref, wdg_ref, wsu_ref, p_ref,
                 gau_ref, adg_ref, asu_ref):
    x = x_ref[...]
    asg = _dotT(x, wsg_ref[...])
    adu = _dotT(x, wdu_ref[...]) + p_ref[0][None, :]
    gau_ref[...] = _pack2(asg, adu)
    adg_ref[...] = _dotT(x, wdg_ref[...])
    asu_ref[...] = _dotT(x, wsu_ref[...])


def _edge_pre_body(ea_ref, ga_ref, gb_ref, weg_ref, pre_ref, st_ref):
    i = pl.program_id(0)
    ga = _unpack_lo(ga_ref[...])
    pre = _dotT(ea_ref[...], weg_ref[...]) + ga + gb_ref[...]
    pre_ref[...] = pre
    s1 = jnp.sum(pre, axis=0)
    s2 = jnp.sum(pre * pre, axis=0)
    blk = jnp.concatenate([s1[None], s2[None], jnp.zeros((6, D), _f32)], 0)

    @pl.when(i == 0)
    def _():
        st_ref[...] = blk

    @pl.when(i > 0)
    def _():
        st_ref[...] += blk


def _edge_fin_body(pre_ref, ea_ref, au_ref, sta_ref, stb_ref, p_ref,
                   nea_ref, sig_ref, msg_ref):
    st = sta_ref[...] + stb_ref[...]
    mean = st[0] / E
    var = st[1] / E - mean * mean
    inv = lax.rsqrt(var + 1e-5)
    xh = (pre_ref[...] - mean[None, :]) * inv[None, :] * p_ref[0][None, :] \
        + p_ref[1][None, :]
    nea = ea_ref[...] + xh * jax.nn.sigmoid(xh)
    sig = jax.nn.sigmoid(nea)
    nea_ref[...] = nea
    sig_ref[...] = sig
    msg_ref[...] = sig * _unpack_hi(au_ref[...])


def _edge_fin_body_b(pre_ref, ea_ref, au_ref, sta_ref, stb_ref, p_ref,
                     nea_in_ref, nea_ref, sig_ref, msg_ref):
    del nea_in_ref
    _edge_fin_body(pre_ref, ea_ref, au_ref, sta_ref, stb_ref, p_ref,
                   nea_ref, sig_ref, msg_ref)


def _node_pre_body(asu_ref, na_ref, nb_ref, ea_ref, eb_ref, pre_ref, st_ref):
    i = pl.program_id(0)
    pre = asu_ref[...] + (na_ref[...] + nb_ref[...]) \
        / (ea_ref[...] + eb_ref[...] + 1e-6)
    pre_ref[...] = pre
    s1 = jnp.sum(pre, axis=0)
    s2 = jnp.sum(pre * pre, axis=0)
    blk = jnp.concatenate([s1[None], s2[None], jnp.zeros((6, D), _f32)], 0)

    @pl.when(i == 0)
    def _():
        st_ref[...] = blk

    @pl.when(i > 0)
    def _():
        st_ref[...] += blk


def _node_fin_body(x_ref, pre_ref, st_ref, p_ref, out_ref):
    st = st_ref[...]
    mean = st[0] / N
    var = st[1] / N - mean * mean
    inv = lax.rsqrt(var + 1e-5)
    xh = (pre_ref[...] - mean[None, :]) * inv[None, :] * p_ref[0][None, :] \
        + p_ref[1][None, :]
    out_ref[...] = x_ref[...] + xh * jax.nn.sigmoid(xh)


def _sc_mesh():
    return plsc.VectorSubcoreMesh(core_axis_name="c", subcore_axis_name="s",
                                  num_cores=NC, num_subcores=NS)


@functools.cache
def _build_sc_gather(ebase, ecount):
    return functools.partial(
        pl.kernel,
        out_type=(jax.ShapeDtypeStruct((ecount, D), jnp.int32),
                  jax.ShapeDtypeStruct((ecount, D), _f32)),
        mesh=_sc_mesh(),
        scratch_types=[
            pltpu.VMEM((2, CH), jnp.int32),
            pltpu.VMEM((2, CH), jnp.int32),
            pltpu.VMEM((2, CH, D), jnp.int32),
            pltpu.VMEM((2, CH, D), _f32),
            pltpu.SemaphoreType.DMA,
            pltpu.SemaphoreType.DMA,
            pltpu.SemaphoreType.DMA,
            pltpu.SemaphoreType.DMA,
        ],
    )(functools.partial(_sc_gather_body, ebase, ecount))


def _sc_gather(ebase, ecount, gtab, adg, row, col):
    return _build_sc_gather(ebase, ecount)(gtab, adg, row, col)


def _sc_gather_body(ebase, ecount, gtab_hbm, adg_hbm, row_hbm, col_hbm,
                    gau_hbm, gb_hbm, idr, idc, abuf, bbuf, *sems):
    wid = lax.axis_index("s") * NC + lax.axis_index("c")
    epw = ecount // NW
    nch = epw // CH

    def start(k, slot):
        base = wid * epw + k * CH
        pltpu.sync_copy(row_hbm.at[pl.ds(ebase + base, CH)], idr.at[slot])
        pltpu.sync_copy(col_hbm.at[pl.ds(ebase + base, CH)], idc.at[slot])
        cp1 = pltpu.async_copy(gtab_hbm.at[idr.at[slot]], abuf.at[slot],
                               sems[2 * slot + 0])
        cp2 = pltpu.async_copy(adg_hbm.at[idc.at[slot]], bbuf.at[slot],
                               sems[2 * slot + 1])
        return cp1, cp2

    def finish(k, slot, cps):
        base = wid * epw + k * CH
        cps[0].wait()
        pltpu.sync_copy(abuf.at[slot], gau_hbm.at[pl.ds(base, CH)])
        cps[1].wait()
        pltpu.sync_copy(bbuf.at[slot], gb_hbm.at[pl.ds(base, CH)])

    def body(j, carry):
        k = j * 2
        cps0 = start(k, 0)
        cps1 = start(k + 1, 1)
        finish(k, 0, cps0)
        finish(k + 1, 1, cps1)
        return carry

    lax.fori_loop(0, nch // 2, body, 0)
    if nch % 2:
        finish(nch - 1, 0, start(nch - 1, 0))


@functools.cache
def _build_sc_scatter(ebase, ecount):
    sch = next(c for c in (320, 160, 80) if (ecount // NS) % c == 0)
    return functools.partial(
        pl.kernel,
        out_type=(jax.ShapeDtypeStruct((N, D), _f32),
                  jax.ShapeDtypeStruct((N, D), _f32)),
        mesh=_sc_mesh(),
        scratch_types=[
            pltpu.VMEM_SHARED((ACC_ROWS, D), _f32),
            pltpu.VMEM((16, D), _f32),
            pltpu.VMEM((sch,), jnp.int32),
            pltpu.VMEM((sch, D), _f32),
        ],
    )(functools.partial(_sc_scatter_body, ebase, ecount, sch))


def _sc_scatter(ebase, ecount, sig, msg, col):
    return _build_sc_scatter(ebase, ecount)(sig, msg, col)


def _sc_scatter_body(ebase, ecount, sch, sig_hbm, msg_hbm, col_hbm,
                     nag_hbm, eag_hbm, acc, zb, lidx, dbuf):
    c = lax.axis_index("c")
    s = lax.axis_index("s")

    zv = jnp.zeros((16,), _f32)
    for i in range(16):
        for g in range(D // 16):
            zb[i, pl.ds(g * 16, 16)] = zv

    def zbody(k, carry):
        pltpu.sync_copy(zb, acc.at[pl.ds(s * ROWS_PER_TILE + k * 16, 16)])
        return carry

    lax.fori_loop(0, ROWS_PER_TILE // 16, zbody, 0)
    plsc.subcore_barrier()

    epw = ecount // NS

    def make_loop(data_hbm):
        def body(k, carry):
            eb = s * epw + k * sch
            pltpu.sync_copy(col_hbm.at[pl.ds(ebase + eb, sch)], lidx)
            pltpu.sync_copy(data_hbm.at[pl.ds(eb, sch)], dbuf)
            pltpu.sync_copy(dbuf, acc.at[lidx], add=True)
            return carry
        return body

    @pl.when(c == 0)
    def _():
        lax.fori_loop(0, epw // sch, make_loop(sig_hbm), 0)

    @pl.when(c == 1)
    def _():
        lax.fori_loop(0, epw // sch, make_loop(msg_hbm), 0)

    plsc.subcore_barrier()

    tail = N - (NS - 1) * ROWS_PER_TILE

    def dump(out_hbm):
        off = s * ROWS_PER_TILE

        @pl.when(s < NS - 1)
        def _():
            pltpu.sync_copy(acc.at[pl.ds(off, ROWS_PER_TILE)],
                            out_hbm.at[pl.ds(off, ROWS_PER_TILE)])

        @pl.when(s == NS - 1)
        def _():
            pltpu.sync_copy(acc.at[pl.ds(off, tail)],
                            out_hbm.at[pl.ds(off, tail)])

    @pl.when(c == 0)
    def _():
        dump(eag_hbm)

    @pl.when(c == 1)
    def _():
        dump(nag_hbm)


def kernel(node_feats, edge_attr, edge_index, Wsg, bsg, Wdg, bdg, Weg, beg,
           g1, b1, Wsu, bsu, Wdu, bdu, g2, b2):
    del bsg, bdg, beg, bsu
    row = edge_index[0]
    col = edge_index[1]
    p_tab = jnp.concatenate([bdu[None], jnp.zeros((7, D), _f32)], 0)
    p_e = jnp.concatenate([g1[None], b1[None], jnp.zeros((6, D), _f32)], 0)
    p_n = jnp.concatenate([g2[None], b2[None], jnp.zeros((6, D), _f32)], 0)

    nb = N // BN_BLK
    eb = E // BE_BLK

    gtab, adg, asu = pl.pallas_call(
        _tables_body,
        grid=(nb,),
        in_specs=[
            pl.BlockSpec((BN_BLK, D), lambda i: (i, 0)),
            pl.BlockSpec((D, D), lambda i: (0, 0)),
            pl.BlockSpec((D, D), lambda i: (0, 0)),
            pl.BlockSpec((D, D), lambda i: (0, 0)),
            pl.BlockSpec((D, D), lambda i: (0, 0)),
            pl.BlockSpec((8, D), lambda i: (0, 0)),
        ],
        out_specs=[
            pl.BlockSpec((BN_BLK, D), lambda i: (i, 0)),
            pl.BlockSpec((BN_BLK, D), lambda i: (i, 0)),
            pl.BlockSpec((BN_BLK, D), lambda i: (i, 0)),
        ],
        out_shape=[
            jax.ShapeDtypeStruct((N, D), jnp.int32),
            jax.ShapeDtypeStruct((N, D), _f32),
            jax.ShapeDtypeStruct((N, D), _f32),
        ],
    )(node_feats, Wsg, Wdu, Wdg, Wsu, p_tab)

    gau1, gb1 = _sc_gather(0, EH1, gtab, adg, row, col)
    gau2, gb2 = _sc_gather(EH1, EH2, gtab, adg, row, col)

    eb1 = EH1 // BE_BLK
    eb2 = EH2 // BE_BLK

    def _tc2(gau, gb, base_blk, nblk, ecount):
        return pl.pallas_call(
            _edge_pre_body,
            grid=(nblk,),
            in_specs=[
                pl.BlockSpec((BE_BLK, D), lambda i, b=base_blk: (i + b, 0)),
                pl.BlockSpec((BE_BLK, D), lambda i: (i, 0)),
                pl.BlockSpec((BE_BLK, D), lambda i: (i, 0)),
                pl.BlockSpec((D, D), lambda i: (0, 0)),
            ],
            out_specs=[
                pl.BlockSpec((BE_BLK, D), lambda i: (i, 0)),
                pl.BlockSpec((8, D), lambda i: (0, 0)),
            ],
            out_shape=[
                jax.ShapeDtypeStruct((ecount, D), _f32),
                jax.ShapeDtypeStruct((8, D), _f32),
            ],
        )(edge_attr, gau, gb, Weg)

    pre1, st1a = _tc2(gau1, gb1, 0, eb1, EH1)
    pre2, st1b = _tc2(gau2, gb2, eb1, eb2, EH2)

    _sml = pl.BlockSpec((8, D), lambda i: (0, 0))

    nea1, sig1, msg1 = pl.pallas_call(
        _edge_fin_body,
        grid=(eb1,),
        in_specs=[
            pl.BlockSpec((BE_BLK, D), lambda i: (i, 0)),
            pl.BlockSpec((BE_BLK, D), lambda i: (i, 0)),
            pl.BlockSpec((BE_BLK, D), lambda i: (i, 0)),
            _sml, _sml, _sml,
        ],
        out_specs=[
            pl.BlockSpec((BE_BLK, D), lambda i: (i, 0)),
            pl.BlockSpec((BE_BLK, D), lambda i: (i, 0)),
            pl.BlockSpec((BE_BLK, D), lambda i: (i, 0)),
        ],
        out_shape=[
            jax.ShapeDtypeStruct((E, D), _f32),
            jax.ShapeDtypeStruct((EH1, D), _f32),
            jax.ShapeDtypeStruct((EH1, D), _f32),
        ],
    )(pre1, edge_attr, gau1, st1a, st1b, p_e)

    nag1, eag1 = _sc_scatter(0, EH1, sig1, msg1, col)

    nea, sig2, msg2 = pl.pallas_call(
        _edge_fin_body_b,
        grid=(eb2,),
        in_specs=[
            pl.BlockSpec((BE_BLK, D), lambda i: (i, 0)),
            pl.BlockSpec((BE_BLK, D), lambda i, b=eb1: (i + b, 0)),
            pl.BlockSpec((BE_BLK, D), lambda i: (i, 0)),
            _sml, _sml, _sml,
            pl.BlockSpec(memory_space=pl.ANY),
        ],
        out_specs=[
            pl.BlockSpec((BE_BLK, D), lambda i, b=eb1: (i + b, 0)),
            pl.BlockSpec((BE_BLK, D), lambda i: (i, 0)),
            pl.BlockSpec((BE_BLK, D), lambda i: (i, 0)),
        ],
        out_shape=[
            jax.ShapeDtypeStruct((E, D), _f32),
            jax.ShapeDtypeStruct((EH2, D), _f32),
            jax.ShapeDtypeStruct((EH2, D), _f32),
        ],
        input_output_aliases={6: 0},
    )(pre2, edge_attr, gau2, st1a, st1b, p_e, nea1)

    nag2, eag2 = _sc_scatter(EH1, EH2, sig2, msg2, col)

    pre_n, st2 = pl.pallas_call(
        _node_pre_body,
        grid=(nb,),
        in_specs=[
            pl.BlockSpec((BN_BLK, D), lambda i: (i, 0)),
            pl.BlockSpec((BN_BLK, D), lambda i: (i, 0)),
            pl.BlockSpec((BN_BLK, D), lambda i: (i, 0)),
            pl.BlockSpec((BN_BLK, D), lambda i: (i, 0)),
            pl.BlockSpec((BN_BLK, D), lambda i: (i, 0)),
        ],
        out_specs=[
            pl.BlockSpec((BN_BLK, D), lambda i: (i, 0)),
            pl.BlockSpec((8, D), lambda i: (0, 0)),
        ],
        out_shape=[
            jax.ShapeDtypeStruct((N, D), _f32),
            jax.ShapeDtypeStruct((8, D), _f32),
        ],
    )(asu, nag1, nag2, eag1, eag2)

    new_node_feats = pl.pallas_call(
        _node_fin_body,
        grid=(nb,),
        in_specs=[
            pl.BlockSpec((BN_BLK, D), lambda i: (i, 0)),
            pl.BlockSpec((BN_BLK, D), lambda i: (i, 0)),
            pl.BlockSpec((8, D), lambda i: (0, 0)),
            pl.BlockSpec((8, D), lambda i: (0, 0)),
        ],
        out_specs=pl.BlockSpec((BN_BLK, D), lambda i: (i, 0)),
        out_shape=jax.ShapeDtypeStruct((N, D), _f32),
    )(node_feats, pre_n, st2, p_n)

    return (new_node_feats, nea)
